# baseline (speedup 1.0000x reference)
import functools

import jax
import jax.numpy as jnp
from jax.experimental import pallas as pl

N_ROWS = 10000
N_COLS = 10000


def _pool_m(indices, values, axis, n):
    seg = indices[1 - axis]
    s = jax.ops.segment_sum(values, seg, num_segments=n)
    c = jax.ops.segment_sum(jnp.ones((values.shape[0], 1), values.dtype), seg, num_segments=n)
    return (s / c)[seg]


def _exch_m(W, b, indices, values):
    r = _pool_m(indices, values, 0, N_COLS)
    c = _pool_m(indices, values, 1, N_ROWS)
    m = jnp.broadcast_to(jnp.mean(values, axis=0, keepdims=True), values.shape)
    x = jnp.concatenate([values, r, c, m], axis=1)
    return jax.nn.leaky_relu(x @ W.T + b, 0.01)


def _deg_m(indices, values):
    flat = values[:, 0]
    values_neg = jnp.where(flat == 0.0, -1.0, flat)
    prod = values_neg * indices[1].astype(jnp.float32)
    _, counts = jnp.unique(prod, return_counts=True, size=prod.shape[0], fill_value=0.0)
    u = jnp.sum(counts > 0)
    idx = jnp.clip(indices[1], 0, u - 1)
    return counts[idx].astype(jnp.float32)


def _mlp_body(em_ref, w1_ref, b1_ref, w2_ref, b2_ref, w3_ref, b3_ref, out_ref):
    h = jnp.maximum(jnp.dot(em_ref[...], w1_ref[...].T, preferred_element_type=jnp.float32) + b1_ref[...], 0.0)
    h = jnp.maximum(jnp.dot(h, w2_ref[...].T, preferred_element_type=jnp.float32) + b2_ref[...], 0.0)
    out_ref[...] = jnp.dot(h, w3_ref[...].T, preferred_element_type=jnp.float32) + b3_ref[...]


def _mlp_pallas(em, W1, b1, W2, b2, W3, b3):
    n = em.shape[0]
    blk = 1000
    grid = n // blk
    return pl.pallas_call(
        _mlp_body,
        grid=(grid,),
        in_specs=[
            pl.BlockSpec((blk, em.shape[1]), lambda i: (i, 0)),
            pl.BlockSpec(W1.shape, lambda i: (0, 0)),
            pl.BlockSpec(b1.shape, lambda i: (0,)),
            pl.BlockSpec(W2.shape, lambda i: (0, 0)),
            pl.BlockSpec(b2.shape, lambda i: (0,)),
            pl.BlockSpec(W3.shape, lambda i: (0, 0)),
            pl.BlockSpec(b3.shape, lambda i: (0,)),
        ],
        out_specs=pl.BlockSpec((blk, W3.shape[0]), lambda i: (i, 0)),
        out_shape=jax.ShapeDtypeStruct((n, W3.shape[0]), jnp.float32),
    )(em, W1, b1, W2, b2, W3, b3)


def kernel(indices, values, embed, ex0_W, ex0_b, ex1_W, ex1_b, ex2_W, ex2_b, ex3_W, ex3_b, ex4_W, ex4_b, ex5_W, ex5_b, ex6_W, ex6_b, ex7_W, ex7_b, ex8_W, ex8_b, cl_W1, cl_b1, cl_W2, cl_b2, cl_W3, cl_b3, vl_W1, vl_b1, vl_W2, vl_b2, vl_W3, vl_b3):
    exW = [ex0_W, ex1_W, ex2_W, ex3_W, ex4_W, ex5_W, ex6_W, ex7_W, ex8_W]
    exb = [ex0_b, ex1_b, ex2_b, ex3_b, ex4_b, ex5_b, ex6_b, ex7_b, ex8_b]
    degree = _deg_m(indices, values)
    flat = values[:, 0]
    new_values = jnp.take(embed, flat.astype(jnp.int32), axis=0)
    v = jnp.concatenate([new_values, degree[:, None]], axis=1)
    for i in range(9):
        v = _exch_m(exW[i], exb[i], indices, v)
    em = jax.ops.segment_sum(v, indices[1], num_segments=N_COLS)
    counts_out = _mlp_pallas(em, cl_W1, cl_b1, cl_W2, cl_b2, cl_W3, cl_b3)
    sf = jnp.stack([jnp.asarray(float(values.shape[0]), jnp.float32), indices[0].max().astype(jnp.float32), indices[1].max().astype(jnp.float32)]) / 100.0
    em2 = jnp.concatenate([em, jnp.broadcast_to(sf[None, :], (em.shape[0], 3))], axis=1)
    pooled = jnp.mean(em2, axis=0)
    h = jnp.maximum(pooled @ vl_W1.T + vl_b1, 0.0)
    h = jnp.maximum(h @ vl_W2.T + vl_b2, 0.0)
    val = h @ vl_W3.T + vl_b3
    value = jnp.mean(jnp.squeeze(val))
    return jnp.concatenate([counts_out.reshape(-1), jnp.atleast_1d(value)])


# speedup vs baseline: 2.0351x; 2.0351x over previous
import functools

import jax
import jax.numpy as jnp
from jax import lax
from jax.experimental import pallas as pl
from jax.experimental.pallas import tpu as pltpu, tpu_sc as plsc

E = 320000
V = 10000
VP = 10240
NBINS = 20480
NC, NS, L = 2, 16, 16
NW = NC * NS
PERW = E // NW
CH = 200
NCH = PERW // CH

_mesh = plsc.VectorSubcoreMesh(core_axis_name="c", subcore_axis_name="s")
_scparams = pltpu.CompilerParams(needs_layout_passes=False)


def _zero_vmem(ref, n):
    def z(i, _):
        ref[pl.ds(i * L, L)] = jnp.zeros((L,), jnp.float32)
        return _
    lax.fori_loop(0, n // L, z, None)


@functools.partial(
    pl.kernel,
    out_type=[
        jax.ShapeDtypeStruct((NC, NBINS), jnp.float32),
        jax.ShapeDtypeStruct((NC, VP), jnp.float32),
        jax.ShapeDtypeStruct((NC, VP), jnp.float32),
        jax.ShapeDtypeStruct((NC, VP), jnp.float32),
        jax.ShapeDtypeStruct((NC, VP), jnp.float32),
    ],
    mesh=_mesh,
    compiler_params=_scparams,
    scratch_types=[
        pltpu.VMEM((PERW,), jnp.int32),
        pltpu.VMEM((PERW,), jnp.int32),
        pltpu.VMEM((PERW,), jnp.float32),
        pltpu.VMEM((NBINS,), jnp.float32),
        pltpu.VMEM((VP,), jnp.float32),
        pltpu.VMEM((VP,), jnp.float32),
        pltpu.VMEM((VP,), jnp.float32),
        pltpu.VMEM((VP,), jnp.float32),
        pltpu.VMEM((NBINS // NS,), jnp.float32),
        pltpu.VMEM((NBINS // NS,), jnp.float32),
        pltpu.VMEM_SHARED((NS, NBINS), jnp.float32),
    ],
)
def _sc_stats(i0_hbm, i1_hbm, f_hbm, hist_hbm, c1_hbm, c0_hbm, n1_hbm, n0_hbm,
              i0_v, i1_v, f_v, hist_v, c1_v, c0_v, n1_v, n0_v, acc_v, tmp_v,
              sh_h):
    cid = lax.axis_index("c")
    sid = lax.axis_index("s")
    wid = cid * NS + sid
    _zero_vmem(hist_v, NBINS)
    _zero_vmem(c1_v, VP)
    _zero_vmem(c0_v, VP)
    _zero_vmem(n1_v, VP)
    _zero_vmem(n0_v, VP)
    base = wid * PERW
    pltpu.sync_copy(i0_hbm.at[pl.ds(base, PERW)], i0_v)
    pltpu.sync_copy(i1_hbm.at[pl.ds(base, PERW)], i1_v)
    pltpu.sync_copy(f_hbm.at[pl.ds(base, PERW)], f_v)
    ones = jnp.ones((L,), jnp.float32)

    def body(j, _):
        i1 = i1_v[pl.ds(j * L, L)]
        i0 = i0_v[pl.ds(j * L, L)]
        f = f_v[pl.ds(j * L, L)]
        binv = jnp.where(f == 0.0, -i1, i1) + 9999
        plsc.addupdate_scatter(hist_v, [binv], ones)
        plsc.addupdate_scatter(c1_v, [i1], ones)
        plsc.addupdate_scatter(c0_v, [i0], ones)
        plsc.addupdate_scatter(n1_v, [i1], f)
        plsc.addupdate_scatter(n0_v, [i0], f)
        return _

    lax.fori_loop(0, PERW // L, body, None)

    def reduce_out(local_v, shared, out_ref, size):
        pltpu.sync_copy(local_v, shared.at[sid, pl.ds(0, size)])
        plsc.subcore_barrier()
        sl = size // NS
        rbase = sid * sl
        pltpu.sync_copy(shared.at[0, pl.ds(rbase, sl)], acc_v.at[pl.ds(0, sl)])

        def red(k, _):
            pltpu.sync_copy(shared.at[k, pl.ds(rbase, sl)], tmp_v.at[pl.ds(0, sl)])

            def addv(i, __):
                acc_v[pl.ds(i * L, L)] = acc_v[pl.ds(i * L, L)] + tmp_v[pl.ds(i * L, L)]
                return __
            lax.fori_loop(0, sl // L, addv, None)
            return _
        lax.fori_loop(1, NS, red, None)
        pltpu.sync_copy(acc_v.at[pl.ds(0, sl)], out_ref.at[cid, pl.ds(rbase, sl)])
        plsc.subcore_barrier()

    reduce_out(hist_v, sh_h, hist_hbm, NBINS)
    reduce_out(c1_v, sh_h, c1_hbm, VP)
    reduce_out(c0_v, sh_h, c0_hbm, VP)
    reduce_out(n1_v, sh_h, n1_hbm, VP)
    reduce_out(n0_v, sh_h, n0_hbm, VP)


@functools.partial(
    pl.kernel,
    out_type=jax.ShapeDtypeStruct((NBINS,), jnp.float32),
    mesh=_mesh,
    compiler_params=_scparams,
    scratch_types=[
        pltpu.VMEM((NBINS,), jnp.float32),
        pltpu.VMEM((NBINS,), jnp.float32),
    ],
)
def _sc_compact(hist_hbm, out_hbm, hist_v, comp_v):
    cid = lax.axis_index("c")
    sid = lax.axis_index("s")

    @pl.when(jnp.logical_and(cid == 0, sid == 0))
    def _():
        pltpu.sync_copy(hist_hbm, hist_v)
        _zero_vmem(comp_v, NBINS)

        def body(j, carry):
            v = hist_v[pl.ds(j * L, L)]
            mask = v > 0.0
            mi = mask.astype(jnp.int32)
            cs = plsc.cumsum(mi)
            pos = jnp.maximum(carry + cs - 1, 0)
            plsc.store_scatter(comp_v, [pos], v, mask=mask)
            return carry + jnp.sum(mi)

        lax.fori_loop(0, NBINS // L, body, jnp.int32(0))
        pltpu.sync_copy(comp_v, out_hbm)


@functools.partial(
    pl.kernel,
    out_type=[
        jax.ShapeDtypeStruct((E,), jnp.float32),
        jax.ShapeDtypeStruct((NC, VP), jnp.float32),
        jax.ShapeDtypeStruct((NC, VP), jnp.float32),
    ],
    mesh=_mesh,
    compiler_params=_scparams,
    scratch_types=[
        pltpu.VMEM((NBINS,), jnp.float32),
        pltpu.VMEM((L,), jnp.int32),
        pltpu.VMEM((PERW,), jnp.int32),
        pltpu.VMEM((PERW,), jnp.int32),
        pltpu.VMEM((PERW,), jnp.float32),
        pltpu.VMEM((VP,), jnp.float32),
        pltpu.VMEM((VP,), jnp.float32),
        pltpu.VMEM((VP // NS,), jnp.float32),
        pltpu.VMEM((VP // NS,), jnp.float32),
        pltpu.VMEM_SHARED((NS, VP), jnp.float32),
    ],
)
def _sc_degree(i1_hbm, i0_hbm, comp_hbm, urep_hbm,
               deg_hbm, dg1_hbm, dg0_hbm,
               comp_v, u_v, i1_v, i0_v, deg_v, dg1_v, dg0_v, acc_v, tmp_v, sh):
    cid = lax.axis_index("c")
    sid = lax.axis_index("s")
    wid = cid * NS + sid
    base = wid * PERW
    pltpu.sync_copy(comp_hbm, comp_v)
    pltpu.sync_copy(urep_hbm, u_v)
    pltpu.sync_copy(i1_hbm.at[pl.ds(base, PERW)], i1_v)
    pltpu.sync_copy(i0_hbm.at[pl.ds(base, PERW)], i0_v)
    _zero_vmem(dg1_v, VP)
    _zero_vmem(dg0_v, VP)

    def body(j, _):
        i1 = i1_v[pl.ds(j * L, L)]
        i0 = i0_v[pl.ds(j * L, L)]
        um = u_v[...]
        ic = jnp.maximum(jnp.minimum(i1, um - 1), 0)
        d = plsc.load_gather(comp_v, [ic])
        deg_v[pl.ds(j * L, L)] = d
        plsc.addupdate_scatter(dg1_v, [i1], d)
        plsc.addupdate_scatter(dg0_v, [i0], d)
        return _

    lax.fori_loop(0, PERW // L, body, None)
    pltpu.sync_copy(deg_v, deg_hbm.at[pl.ds(base, PERW)])

    def reduce_out(local_v, out_ref):
        pltpu.sync_copy(local_v, sh.at[sid])
        plsc.subcore_barrier()
        sl = VP // NS
        rbase = sid * sl
        pltpu.sync_copy(sh.at[0, pl.ds(rbase, sl)], acc_v)

        def red(k, _):
            pltpu.sync_copy(sh.at[k, pl.ds(rbase, sl)], tmp_v)

            def addv(i, __):
                acc_v[pl.ds(i * L, L)] = acc_v[pl.ds(i * L, L)] + tmp_v[pl.ds(i * L, L)]
                return __
            lax.fori_loop(0, sl // L, addv, None)
            return _
        lax.fori_loop(1, NS, red, None)
        pltpu.sync_copy(acc_v, out_ref.at[cid, pl.ds(rbase, sl)])
        plsc.subcore_barrier()

    reduce_out(dg1_v, dg1_hbm)
    reduce_out(dg0_v, dg0_hbm)


VH = VP // 2
TAB = VH + 128
RPT = TAB // NS
CHS = 160
PERC = E // NS


@functools.lru_cache(maxsize=None)
def _sc_segsum(D):
    @functools.partial(
        pl.kernel,
        out_type=jax.ShapeDtypeStruct((NC * TAB, D), jnp.float32),
        mesh=_mesh,
        compiler_params=_scparams,
        scratch_types=[
            pltpu.VMEM((CHS,), jnp.int32),
            pltpu.VMEM((CHS,), jnp.int32),
            pltpu.VMEM((CHS, D), jnp.float32),
            pltpu.VMEM((RPT, D), jnp.float32),
            pltpu.VMEM_SHARED((TAB, D), jnp.float32),
        ],
    )
    def k(x_hbm, seg_hbm, out_hbm, idx_v, lidx_v, x_v, zbuf_v, table):
        cid = lax.axis_index("c")
        sid = lax.axis_index("s")

        def z2(i, _):
            def z3(j, __):
                zbuf_v[i, pl.ds(j * L, L)] = jnp.zeros((L,), jnp.float32)
                return __
            lax.fori_loop(0, D // L, z3, None)
            return _
        lax.fori_loop(0, RPT, z2, None)
        pltpu.sync_copy(zbuf_v, table.at[pl.ds(sid * RPT, RPT)])
        plsc.subcore_barrier()
        lo = cid * VH

        def body(j, _):
            base = sid * PERC + j * CHS
            pltpu.sync_copy(seg_hbm.at[pl.ds(base, CHS)], idx_v)
            pltpu.sync_copy(x_hbm.at[pl.ds(base, CHS)], x_v)

            def tr(t, __):
                s = idx_v[pl.ds(t * L, L)] - lo
                oob = jnp.logical_or(s < 0, s >= VH)
                lidx_v[pl.ds(t * L, L)] = jnp.where(oob, VH, s)
                return __
            lax.fori_loop(0, CHS // L, tr, None)
            pltpu.sync_copy(x_v, table.at[lidx_v], add=True)
            return _
        lax.fori_loop(0, PERC // CHS, body, None)
        plsc.subcore_barrier()
        pltpu.sync_copy(table.at[pl.ds(sid * RPT, RPT)], zbuf_v)
        pltpu.sync_copy(zbuf_v, out_hbm.at[pl.ds(cid * TAB + sid * RPT, RPT)])

    return k


@functools.partial(
    pl.kernel,
    out_type=jax.ShapeDtypeStruct((E, 128), jnp.float32),
    mesh=_mesh,
    compiler_params=_scparams,
    scratch_types=[
        pltpu.VMEM((CH,), jnp.int32),
        pltpu.VMEM((CH, 128), jnp.float32),
        pltpu.SemaphoreType.DMA,
    ],
)
def _sc_gather(table_hbm, idx_hbm, out_hbm, idx_v, rows_v, sem):
    cid = lax.axis_index("c")
    sid = lax.axis_index("s")
    wid = cid * NS + sid

    def body(j, _):
        base = wid * PERW + j * CH
        pltpu.sync_copy(idx_hbm.at[pl.ds(base, CH)], idx_v)
        pltpu.async_copy(table_hbm.at[idx_v], rows_v, sem).wait()
        pltpu.sync_copy(rows_v, out_hbm.at[pl.ds(base, CH)])
        return _
    lax.fori_loop(0, NCH, body, None)


def _tc_prep(histp, c1p, c0p, n1p, n0p):
    def body(h_ref, c1_ref, c0_ref, n1_ref, n0_ref,
             hist_ref, urep_ref, c1i_ref, c0i_ref,
             c1r_ref, c0r_ref, n1r_ref, n0r_ref):
        h = h_ref[0] + h_ref[1]
        hist_ref[0, :] = h
        u = jnp.sum((h > 0.0).astype(jnp.int32))
        urep_ref[...] = jnp.full((1, L), u, jnp.int32)
        c1r_ref[0, :] = c1_ref[0] + c1_ref[1]
        c0r_ref[0, :] = c0_ref[0] + c0_ref[1]
        n1r_ref[0, :] = n1_ref[0] + n1_ref[1]
        n0r_ref[0, :] = n0_ref[0] + n0_ref[1]
        c1i_ref[0, :] = 1.0 / jnp.maximum(c1_ref[0] + c1_ref[1], 1.0)
        c0i_ref[0, :] = 1.0 / jnp.maximum(c0_ref[0] + c0_ref[1], 1.0)

    return pl.pallas_call(
        body,
        out_shape=[
            jax.ShapeDtypeStruct((1, NBINS), jnp.float32),
            jax.ShapeDtypeStruct((1, L), jnp.int32),
            jax.ShapeDtypeStruct((1, VP), jnp.float32),
            jax.ShapeDtypeStruct((1, VP), jnp.float32),
            jax.ShapeDtypeStruct((1, VP), jnp.float32),
            jax.ShapeDtypeStruct((1, VP), jnp.float32),
            jax.ShapeDtypeStruct((1, VP), jnp.float32),
            jax.ShapeDtypeStruct((1, VP), jnp.float32),
        ],
    )(histp, c1p, c0p, n1p, n0p)


def _tc_prep2(d1p, d0p):
    def body(a_ref, b_ref, o1_ref, o0_ref):
        o1_ref[0, :] = a_ref[0] + a_ref[1]
        o0_ref[0, :] = b_ref[0] + b_ref[1]

    return pl.pallas_call(
        body,
        out_shape=[
            jax.ShapeDtypeStruct((1, VP), jnp.float32),
            jax.ShapeDtypeStruct((1, VP), jnp.float32),
        ],
    )(d1p, d0p)


def _tc_max(i0r, i1r):
    nb = i0r.shape[0]

    def body(a_ref, b_ref, m0_ref, m1_ref):
        i = pl.program_id(0)

        @pl.when(i == 0)
        def _():
            m0_ref[...] = jnp.zeros((1, 1), jnp.float32)
            m1_ref[...] = jnp.zeros((1, 1), jnp.float32)
        bm0 = jnp.max(a_ref[...]).astype(jnp.float32)
        bm1 = jnp.max(b_ref[...]).astype(jnp.float32)
        m0_ref[...] = jnp.maximum(m0_ref[...], jnp.full((1, 1), bm0, jnp.float32))
        m1_ref[...] = jnp.maximum(m1_ref[...], jnp.full((1, 1), bm1, jnp.float32))

    return pl.pallas_call(
        body,
        grid=(nb,),
        in_specs=[
            pl.BlockSpec((1, 1, i0r.shape[2]), lambda i: (i, 0, 0)),
            pl.BlockSpec((1, 1, i1r.shape[2]), lambda i: (i, 0, 0)),
        ],
        out_specs=[
            pl.BlockSpec((1, 1), lambda i: (0, 0)),
            pl.BlockSpec((1, 1), lambda i: (0, 0)),
        ],
        out_shape=[
            jax.ShapeDtypeStruct((1, 1), jnp.float32),
            jax.ShapeDtypeStruct((1, 1), jnp.float32),
        ],
    )(i0r, i1r)


def _tc_small0(embed, Ar, wrd, Ac, wcd, c1col, c0col, n1col, n0col, d1col, d0col):
    blk = 2048

    def body(e_ref, ar_ref, wrd_ref, ac_ref, wcd_ref,
             c1_ref, c0_ref, n1_ref, n0_ref, d1_ref, d0_ref,
             rq_ref, cq_ref, sn_ref, sd_ref):
        i = pl.program_id(0)
        e0 = e_ref[0:1, :]
        de = e_ref[1:2, :] - e0
        p0r = jnp.dot(e0, ar_ref[...].T, preferred_element_type=jnp.float32)
        pdr = jnp.dot(de, ar_ref[...].T, preferred_element_type=jnp.float32)
        p0c = jnp.dot(e0, ac_ref[...].T, preferred_element_type=jnp.float32)
        pdc = jnp.dot(de, ac_ref[...].T, preferred_element_type=jnp.float32)
        c1 = c1_ref[...]
        c0 = c0_ref[...]
        n1 = n1_ref[...]
        n0 = n0_ref[...]
        d1 = d1_ref[...]
        d0 = d0_ref[...]
        rq_ref[...] = (c1 * p0r + n1 * pdr + d1 * wrd_ref[...]) / jnp.maximum(c1, 1.0)
        cq_ref[...] = (c0 * p0c + n0 * pdc + d0 * wcd_ref[...]) / jnp.maximum(c0, 1.0)

        @pl.when(i == 0)
        def _():
            sn_ref[...] = jnp.zeros((1, 1), jnp.float32)
            sd_ref[...] = jnp.zeros((1, 1), jnp.float32)
        sn_ref[...] += jnp.sum(n1, keepdims=True).reshape(1, 1)
        sd_ref[...] += jnp.sum(d1, keepdims=True).reshape(1, 1)

    return pl.pallas_call(
        body,
        grid=(VP // blk,),
        in_specs=[
            pl.BlockSpec((2, 128), lambda i: (0, 0)),
            pl.BlockSpec((128, 128), lambda i: (0, 0)),
            pl.BlockSpec((1, 128), lambda i: (0, 0)),
            pl.BlockSpec((128, 128), lambda i: (0, 0)),
            pl.BlockSpec((1, 128), lambda i: (0, 0)),
            pl.BlockSpec((blk, 1), lambda i: (i, 0)),
            pl.BlockSpec((blk, 1), lambda i: (i, 0)),
            pl.BlockSpec((blk, 1), lambda i: (i, 0)),
            pl.BlockSpec((blk, 1), lambda i: (i, 0)),
            pl.BlockSpec((blk, 1), lambda i: (i, 0)),
            pl.BlockSpec((blk, 1), lambda i: (i, 0)),
        ],
        out_specs=[
            pl.BlockSpec((blk, 128), lambda i: (i, 0)),
            pl.BlockSpec((blk, 128), lambda i: (i, 0)),
            pl.BlockSpec((1, 1), lambda i: (0, 0)),
            pl.BlockSpec((1, 1), lambda i: (0, 0)),
        ],
        out_shape=[
            jax.ShapeDtypeStruct((VP, 128), jnp.float32),
            jax.ShapeDtypeStruct((VP, 128), jnp.float32),
            jax.ShapeDtypeStruct((1, 1), jnp.float32),
            jax.ShapeDtypeStruct((1, 1), jnp.float32),
        ],
    )(embed, Ar, wrd, Ac, wcd, c1col, c0col, n1col, n0col, d1col, d0col)


def _tc_big0(flat2, deg2, G1, G0, embed, Av, wvd, Am, wmd, b2, Sn, Sd):
    blk = 2000

    def body(f_ref, d_ref, g1_ref, g0_ref, e_ref, av_ref, wvd_ref,
             am_ref, wmd_ref, b_ref, sn_ref, sd_ref, o_ref):
        e0 = e_ref[0:1, :]
        de = e_ref[1:2, :] - e0
        q0 = jnp.dot(e0, av_ref[...].T, preferred_element_type=jnp.float32)
        qd = jnp.dot(de, av_ref[...].T, preferred_element_type=jnp.float32)
        p0m = jnp.dot(e0, am_ref[...].T, preferred_element_type=jnp.float32)
        pdm = jnp.dot(de, am_ref[...].T, preferred_element_type=jnp.float32)
        mvec = (p0m + (sn_ref[...] * (1.0 / E)) * pdm
                + (sd_ref[...] * (1.0 / E)) * wmd_ref[...] + b_ref[...])
        pre = (q0 + f_ref[...] * qd + d_ref[...] * wvd_ref[...]
               + g1_ref[...] + g0_ref[...] + mvec)
        o_ref[...] = jnp.where(pre >= 0.0, pre, 0.01 * pre)

    return pl.pallas_call(
        body,
        grid=(E // blk,),
        in_specs=[
            pl.BlockSpec((blk, 1), lambda i: (i, 0)),
            pl.BlockSpec((blk, 1), lambda i: (i, 0)),
            pl.BlockSpec((blk, 128), lambda i: (i, 0)),
            pl.BlockSpec((blk, 128), lambda i: (i, 0)),
            pl.BlockSpec((2, 128), lambda i: (0, 0)),
            pl.BlockSpec((128, 128), lambda i: (0, 0)),
            pl.BlockSpec((1, 128), lambda i: (0, 0)),
            pl.BlockSpec((128, 128), lambda i: (0, 0)),
            pl.BlockSpec((1, 128), lambda i: (0, 0)),
            pl.BlockSpec((1, 128), lambda i: (0, 0)),
            pl.BlockSpec((1, 1), lambda i: (0, 0)),
            pl.BlockSpec((1, 1), lambda i: (0, 0)),
        ],
        out_specs=pl.BlockSpec((blk, 128), lambda i: (i, 0)),
        out_shape=jax.ShapeDtypeStruct((E, 128), jnp.float32),
    )(flat2, deg2, G1, G0, embed, Av, wvd, Am, wmd, b2, Sn, Sd)


def _tc_small(Rp, Cp, c1col, c0col, Wr, Wc):
    D = Rp.shape[1]
    blk = 2048

    def body(r_ref, c_ref, c1_ref, c0_ref, wr_ref, wc_ref,
             rq_ref, cq_ref, cs_ref):
        i = pl.program_id(0)
        Rs = r_ref[...]
        Cs = c_ref[...]
        rq_ref[...] = jnp.dot(Rs, wr_ref[...].T, preferred_element_type=jnp.float32) * c1_ref[...]
        cq_ref[...] = jnp.dot(Cs, wc_ref[...].T, preferred_element_type=jnp.float32) * c0_ref[...]

        @pl.when(i == 0)
        def _():
            cs_ref[...] = jnp.zeros((1, D), jnp.float32)
        cs_ref[...] += jnp.sum(Rs, axis=0, keepdims=True)

    return pl.pallas_call(
        body,
        grid=(VP // blk,),
        in_specs=[
            pl.BlockSpec((blk, D), lambda i: (i, 0)),
            pl.BlockSpec((blk, D), lambda i: (i, 0)),
            pl.BlockSpec((blk, 1), lambda i: (i, 0)),
            pl.BlockSpec((blk, 1), lambda i: (i, 0)),
            pl.BlockSpec(Wr.shape, lambda i: (0, 0)),
            pl.BlockSpec(Wc.shape, lambda i: (0, 0)),
        ],
        out_specs=[
            pl.BlockSpec((blk, 128), lambda i: (i, 0)),
            pl.BlockSpec((blk, 128), lambda i: (i, 0)),
            pl.BlockSpec((1, D), lambda i: (0, 0)),
        ],
        out_shape=[
            jax.ShapeDtypeStruct((VP, 128), jnp.float32),
            jax.ShapeDtypeStruct((VP, 128), jnp.float32),
            jax.ShapeDtypeStruct((1, D), jnp.float32),
        ],
    )(Rp, Cp, c1col, c0col, Wr, Wc)


def _tc_big(v, G1, G0, Wv, colsum, Wm, b2):
    D = v.shape[1]
    blk = 2000

    def body(v_ref, g1_ref, g0_ref, wv_ref, cs_ref, wm_ref, b_ref, o_ref):
        mvec = jnp.dot(cs_ref[...] * (1.0 / E), wm_ref[...].T,
                       preferred_element_type=jnp.float32) + b_ref[...]
        pre = (jnp.dot(v_ref[...], wv_ref[...].T, preferred_element_type=jnp.float32)
               + g1_ref[...] + g0_ref[...] + mvec)
        o_ref[...] = jnp.where(pre >= 0.0, pre, 0.01 * pre)

    return pl.pallas_call(
        body,
        grid=(E // blk,),
        in_specs=[
            pl.BlockSpec((blk, D), lambda i: (i, 0)),
            pl.BlockSpec((blk, 128), lambda i: (i, 0)),
            pl.BlockSpec((blk, 128), lambda i: (i, 0)),
            pl.BlockSpec(Wv.shape, lambda i: (0, 0)),
            pl.BlockSpec((1, D), lambda i: (0, 0)),
            pl.BlockSpec(Wm.shape, lambda i: (0, 0)),
            pl.BlockSpec((1, 128), lambda i: (0, 0)),
        ],
        out_specs=pl.BlockSpec((blk, 128), lambda i: (i, 0)),
        out_shape=jax.ShapeDtypeStruct((E, 128), jnp.float32),
    )(v, G1, G0, Wv, colsum, Wm, b2)


def _tc_final(emp, W1, b1, W2, b2, W3p, b3p):
    blk = 2048

    def body(e_ref, w1_ref, b1_ref, w2_ref, b2_ref, w3_ref, b3_ref,
             lg_ref, cs_ref):
        i = pl.program_id(0)
        em = e_ref[...]
        h = jnp.maximum(jnp.dot(em, w1_ref[...].T, preferred_element_type=jnp.float32) + b1_ref[...], 0.0)
        h = jnp.maximum(jnp.dot(h, w2_ref[...].T, preferred_element_type=jnp.float32) + b2_ref[...], 0.0)
        lg_ref[...] = jnp.dot(h, w3_ref[...].T, preferred_element_type=jnp.float32) + b3_ref[...]

        @pl.when(i == 0)
        def _():
            cs_ref[...] = jnp.zeros((1, 128), jnp.float32)
        cs_ref[...] += jnp.sum(em, axis=0, keepdims=True)

    return pl.pallas_call(
        body,
        grid=(VP // blk,),
        in_specs=[
            pl.BlockSpec((blk, 128), lambda i: (i, 0)),
            pl.BlockSpec(W1.shape, lambda i: (0, 0)),
            pl.BlockSpec((1, 128), lambda i: (0, 0)),
            pl.BlockSpec(W2.shape, lambda i: (0, 0)),
            pl.BlockSpec((1, 128), lambda i: (0, 0)),
            pl.BlockSpec(W3p.shape, lambda i: (0, 0)),
            pl.BlockSpec((1, 128), lambda i: (0, 0)),
        ],
        out_specs=[
            pl.BlockSpec((blk, 128), lambda i: (i, 0)),
            pl.BlockSpec((1, 128), lambda i: (0, 0)),
        ],
        out_shape=[
            jax.ShapeDtypeStruct((VP, 128), jnp.float32),
            jax.ShapeDtypeStruct((1, 128), jnp.float32),
        ],
    )(emp, W1, b1, W2, b2, W3p, b3p)


def _tc_value(emcol, sfp, W1p, b1, W2, b2, W3p, b3p):
    def body(ec_ref, sf_ref, w1_ref, b1_ref, w2_ref, b2_ref, w3_ref, b3_ref, o_ref):
        x = jnp.concatenate([ec_ref[...] * (1.0 / V), sf_ref[...]], axis=1)
        h = jnp.maximum(jnp.dot(x, w1_ref[...].T, preferred_element_type=jnp.float32) + b1_ref[...], 0.0)
        h = jnp.maximum(jnp.dot(h, w2_ref[...].T, preferred_element_type=jnp.float32) + b2_ref[...], 0.0)
        o_ref[...] = jnp.dot(h, w3_ref[...].T, preferred_element_type=jnp.float32) + b3_ref[...]

    return pl.pallas_call(
        body,
        out_shape=jax.ShapeDtypeStruct((1, 128), jnp.float32),
    )(emcol, sfp, W1p, b1, W2, b2, W3p, b3p)


def kernel(indices, values, embed,
           ex0_W, ex0_b, ex1_W, ex1_b, ex2_W, ex2_b, ex3_W, ex3_b, ex4_W, ex4_b,
           ex5_W, ex5_b, ex6_W, ex6_b, ex7_W, ex7_b, ex8_W, ex8_b,
           cl_W1, cl_b1, cl_W2, cl_b2, cl_W3, cl_b3,
           vl_W1, vl_b1, vl_W2, vl_b2, vl_W3, vl_b3):
    exW = [ex0_W, ex1_W, ex2_W, ex3_W, ex4_W, ex5_W, ex6_W, ex7_W, ex8_W]
    exb = [ex0_b, ex1_b, ex2_b, ex3_b, ex4_b, ex5_b, ex6_b, ex7_b, ex8_b]

    ind0 = indices[0].astype(jnp.int32)
    ind1 = indices[1].astype(jnp.int32)
    flat = values[:, 0]

    histp, c1p, c0p, n1p, n0p = _sc_stats(ind0, ind1, flat)
    hist2, urep2, c1i, c0i, c1r, c0r, n1r, n0r = _tc_prep(histp, c1p, c0p, n1p, n0p)
    compact = _sc_compact(hist2.reshape(NBINS))
    degree, dg1p, dg0p = _sc_degree(ind1, ind0, compact, urep2.reshape(L))
    dg1, dg0 = _tc_prep2(dg1p, dg0p)
    m0, m1 = _tc_max(ind0.reshape(E // 2000, 1, 2000), ind1.reshape(E // 2000, 1, 2000))

    c1col = c1i.reshape(VP, 1)
    c0col = c0i.reshape(VP, 1)

    W0 = exW[0]
    Wv0, Wr0, Wc0, Wm0 = W0[:, :129], W0[:, 129:258], W0[:, 258:387], W0[:, 387:516]
    Av, wvd = Wv0[:, :128], Wv0[:, 128].reshape(1, 128)
    Ar, wrd = Wr0[:, :128], Wr0[:, 128].reshape(1, 128)
    Ac, wcd = Wc0[:, :128], Wc0[:, 128].reshape(1, 128)
    Am, wmd = Wm0[:, :128], Wm0[:, 128].reshape(1, 128)
    Rq, Cq, Sn, Sd = _tc_small0(
        embed, Ar, wrd, Ac, wcd,
        c1r.reshape(VP, 1), c0r.reshape(VP, 1),
        n1r.reshape(VP, 1), n0r.reshape(VP, 1),
        dg1.reshape(VP, 1), dg0.reshape(VP, 1))
    G1 = _sc_gather(Rq, ind1)
    G0 = _sc_gather(Cq, ind0)
    v = _tc_big0(flat.reshape(E, 1), degree.reshape(E, 1), G1, G0, embed,
                 Av, wvd, Am, wmd, exb[0].reshape(1, 128), Sn, Sd)

    for i in range(1, 9):
        W = exW[i]
        Wv, Wr, Wc, Wm = W[:, :128], W[:, 128:256], W[:, 256:384], W[:, 384:512]
        seg = _sc_segsum(128)
        Rp = seg(v, ind1)
        Cp = seg(v, ind0)
        Rsum = jnp.concatenate([Rp[:VH], Rp[TAB:TAB + VH]], axis=0)
        Csum = jnp.concatenate([Cp[:VH], Cp[TAB:TAB + VH]], axis=0)
        Rq, Cq, colsum = _tc_small(Rsum, Csum, c1col, c0col, Wr, Wc)
        G1 = _sc_gather(Rq, ind1)
        G0 = _sc_gather(Cq, ind0)
        v = _tc_big(v, G1, G0, Wv, colsum, Wm, exb[i].reshape(1, 128))

    emp = _sc_segsum(128)(v, ind1)
    em = jnp.concatenate([emp[:VH], emp[TAB:TAB + VH]], axis=0)
    W3p = jnp.zeros((128, 128), jnp.float32).at[:2, :].set(cl_W3)
    b3p = jnp.zeros((1, 128), jnp.float32).at[0, :2].set(cl_b3)
    logits, emcol = _tc_final(em, cl_W1, cl_b1.reshape(1, 128),
                              cl_W2, cl_b2.reshape(1, 128), W3p, b3p)

    sfp = jnp.zeros((1, 16), jnp.float32)
    sfp = sfp.at[0, 0].set(float(E) / 100.0)
    sfp = sfp.at[0, 1].set(m0[0, 0] / 100.0)
    sfp = sfp.at[0, 2].set(m1[0, 0] / 100.0)
    vW1p = jnp.zeros((128, 144), jnp.float32)
    vW1p = vW1p.at[:, :128].set(vl_W1[:, :128])
    vW1p = vW1p.at[:, 128:131].set(vl_W1[:, 128:131])
    vW3p = jnp.zeros((128, 128), jnp.float32).at[:1, :].set(vl_W3)
    vb3p = jnp.zeros((1, 128), jnp.float32).at[0, :1].set(vl_b3)
    val = _tc_value(emcol, sfp, vW1p, vl_b1.reshape(1, 128),
                    vl_W2, vl_b2.reshape(1, 128), vW3p, vb3p)

    counts_out = logits[:V, :2].reshape(-1)
    return jnp.concatenate([counts_out, val[0, :1]])


# speedup vs baseline: 2.2066x; 1.0843x over previous
import functools

import jax
import jax.numpy as jnp
from jax import lax
from jax.experimental import pallas as pl
from jax.experimental.pallas import tpu as pltpu, tpu_sc as plsc

E = 320000
V = 10000
VP = 10240
NBINS = 20480
NC, NS, L = 2, 16, 16
NW = NC * NS
PERW = E // NW
CH = 200
NCH = PERW // CH

_mesh = plsc.VectorSubcoreMesh(core_axis_name="c", subcore_axis_name="s")
_scparams = pltpu.CompilerParams(needs_layout_passes=False)


def _zero_vmem(ref, n):
    def z(i, _):
        ref[pl.ds(i * L, L)] = jnp.zeros((L,), jnp.float32)
        return _
    lax.fori_loop(0, n // L, z, None)


@functools.partial(
    pl.kernel,
    out_type=[
        jax.ShapeDtypeStruct((NC, NBINS), jnp.float32),
        jax.ShapeDtypeStruct((NC, VP), jnp.float32),
        jax.ShapeDtypeStruct((NC, VP), jnp.float32),
        jax.ShapeDtypeStruct((NC, VP), jnp.float32),
        jax.ShapeDtypeStruct((NC, VP), jnp.float32),
    ],
    mesh=_mesh,
    compiler_params=_scparams,
    scratch_types=[
        pltpu.VMEM((PERW,), jnp.int32),
        pltpu.VMEM((PERW,), jnp.int32),
        pltpu.VMEM((PERW,), jnp.float32),
        pltpu.VMEM((NBINS,), jnp.float32),
        pltpu.VMEM((VP,), jnp.float32),
        pltpu.VMEM((VP,), jnp.float32),
        pltpu.VMEM((VP,), jnp.float32),
        pltpu.VMEM((VP,), jnp.float32),
        pltpu.VMEM((NBINS // NS,), jnp.float32),
        pltpu.VMEM((NBINS // NS,), jnp.float32),
        pltpu.VMEM_SHARED((NS, NBINS), jnp.float32),
    ],
)
def _sc_stats(i0_hbm, i1_hbm, f_hbm, hist_hbm, c1_hbm, c0_hbm, n1_hbm, n0_hbm,
              i0_v, i1_v, f_v, hist_v, c1_v, c0_v, n1_v, n0_v, acc_v, tmp_v,
              sh_h):
    cid = lax.axis_index("c")
    sid = lax.axis_index("s")
    wid = cid * NS + sid
    _zero_vmem(hist_v, NBINS)
    _zero_vmem(c1_v, VP)
    _zero_vmem(c0_v, VP)
    _zero_vmem(n1_v, VP)
    _zero_vmem(n0_v, VP)
    base = wid * PERW
    pltpu.sync_copy(i0_hbm.at[pl.ds(base, PERW)], i0_v)
    pltpu.sync_copy(i1_hbm.at[pl.ds(base, PERW)], i1_v)
    pltpu.sync_copy(f_hbm.at[pl.ds(base, PERW)], f_v)
    ones = jnp.ones((L,), jnp.float32)

    def body(j, _):
        i1 = i1_v[pl.ds(j * L, L)]
        i0 = i0_v[pl.ds(j * L, L)]
        f = f_v[pl.ds(j * L, L)]
        binv = jnp.where(f == 0.0, -i1, i1) + 9999
        plsc.addupdate_scatter(hist_v, [binv], ones)
        plsc.addupdate_scatter(c1_v, [i1], ones)
        plsc.addupdate_scatter(c0_v, [i0], ones)
        plsc.addupdate_scatter(n1_v, [i1], f)
        plsc.addupdate_scatter(n0_v, [i0], f)
        return _

    lax.fori_loop(0, PERW // L, body, None)

    def reduce_out(local_v, shared, out_ref, size):
        pltpu.sync_copy(local_v, shared.at[sid, pl.ds(0, size)])
        plsc.subcore_barrier()
        sl = size // NS
        rbase = sid * sl
        pltpu.sync_copy(shared.at[0, pl.ds(rbase, sl)], acc_v.at[pl.ds(0, sl)])

        def red(k, _):
            pltpu.sync_copy(shared.at[k, pl.ds(rbase, sl)], tmp_v.at[pl.ds(0, sl)])

            def addv(i, __):
                acc_v[pl.ds(i * L, L)] = acc_v[pl.ds(i * L, L)] + tmp_v[pl.ds(i * L, L)]
                return __
            lax.fori_loop(0, sl // L, addv, None)
            return _
        lax.fori_loop(1, NS, red, None)
        pltpu.sync_copy(acc_v.at[pl.ds(0, sl)], out_ref.at[cid, pl.ds(rbase, sl)])
        plsc.subcore_barrier()

    reduce_out(hist_v, sh_h, hist_hbm, NBINS)
    reduce_out(c1_v, sh_h, c1_hbm, VP)
    reduce_out(c0_v, sh_h, c0_hbm, VP)
    reduce_out(n1_v, sh_h, n1_hbm, VP)
    reduce_out(n0_v, sh_h, n0_hbm, VP)


@functools.partial(
    pl.kernel,
    out_type=jax.ShapeDtypeStruct((NBINS,), jnp.float32),
    mesh=_mesh,
    compiler_params=_scparams,
    scratch_types=[
        pltpu.VMEM((NBINS,), jnp.float32),
        pltpu.VMEM((NBINS,), jnp.float32),
    ],
)
def _sc_compact(hist_hbm, out_hbm, hist_v, comp_v):
    cid = lax.axis_index("c")
    sid = lax.axis_index("s")

    @pl.when(jnp.logical_and(cid == 0, sid == 0))
    def _():
        pltpu.sync_copy(hist_hbm, hist_v)
        _zero_vmem(comp_v, NBINS)

        def body(j, carry):
            v = hist_v[pl.ds(j * L, L)]
            mask = v > 0.0
            mi = mask.astype(jnp.int32)
            cs = plsc.cumsum(mi)
            pos = jnp.maximum(carry + cs - 1, 0)
            plsc.store_scatter(comp_v, [pos], v, mask=mask)
            return carry + jnp.sum(mi)

        lax.fori_loop(0, NBINS // L, body, jnp.int32(0))
        pltpu.sync_copy(comp_v, out_hbm)


@functools.partial(
    pl.kernel,
    out_type=[
        jax.ShapeDtypeStruct((E,), jnp.float32),
        jax.ShapeDtypeStruct((NC, VP), jnp.float32),
        jax.ShapeDtypeStruct((NC, VP), jnp.float32),
    ],
    mesh=_mesh,
    compiler_params=_scparams,
    scratch_types=[
        pltpu.VMEM((NBINS,), jnp.float32),
        pltpu.VMEM((L,), jnp.int32),
        pltpu.VMEM((PERW,), jnp.int32),
        pltpu.VMEM((PERW,), jnp.int32),
        pltpu.VMEM((PERW,), jnp.float32),
        pltpu.VMEM((VP,), jnp.float32),
        pltpu.VMEM((VP,), jnp.float32),
        pltpu.VMEM((VP // NS,), jnp.float32),
        pltpu.VMEM((VP // NS,), jnp.float32),
        pltpu.VMEM_SHARED((NS, VP), jnp.float32),
    ],
)
def _sc_degree(i1_hbm, i0_hbm, comp_hbm, urep_hbm,
               deg_hbm, dg1_hbm, dg0_hbm,
               comp_v, u_v, i1_v, i0_v, deg_v, dg1_v, dg0_v, acc_v, tmp_v, sh):
    cid = lax.axis_index("c")
    sid = lax.axis_index("s")
    wid = cid * NS + sid
    base = wid * PERW
    pltpu.sync_copy(comp_hbm, comp_v)
    pltpu.sync_copy(urep_hbm, u_v)
    pltpu.sync_copy(i1_hbm.at[pl.ds(base, PERW)], i1_v)
    pltpu.sync_copy(i0_hbm.at[pl.ds(base, PERW)], i0_v)
    _zero_vmem(dg1_v, VP)
    _zero_vmem(dg0_v, VP)

    def body(j, _):
        i1 = i1_v[pl.ds(j * L, L)]
        i0 = i0_v[pl.ds(j * L, L)]
        um = u_v[...]
        ic = jnp.maximum(jnp.minimum(i1, um - 1), 0)
        d = plsc.load_gather(comp_v, [ic])
        deg_v[pl.ds(j * L, L)] = d
        plsc.addupdate_scatter(dg1_v, [i1], d)
        plsc.addupdate_scatter(dg0_v, [i0], d)
        return _

    lax.fori_loop(0, PERW // L, body, None)
    pltpu.sync_copy(deg_v, deg_hbm.at[pl.ds(base, PERW)])

    def reduce_out(local_v, out_ref):
        pltpu.sync_copy(local_v, sh.at[sid])
        plsc.subcore_barrier()
        sl = VP // NS
        rbase = sid * sl
        pltpu.sync_copy(sh.at[0, pl.ds(rbase, sl)], acc_v)

        def red(k, _):
            pltpu.sync_copy(sh.at[k, pl.ds(rbase, sl)], tmp_v)

            def addv(i, __):
                acc_v[pl.ds(i * L, L)] = acc_v[pl.ds(i * L, L)] + tmp_v[pl.ds(i * L, L)]
                return __
            lax.fori_loop(0, sl // L, addv, None)
            return _
        lax.fori_loop(1, NS, red, None)
        pltpu.sync_copy(acc_v, out_ref.at[cid, pl.ds(rbase, sl)])
        plsc.subcore_barrier()

    reduce_out(dg1_v, dg1_hbm)
    reduce_out(dg0_v, dg0_hbm)


VH = VP // 2
TAB = VH + 128
RPT = TAB // NS
CHS = 160
PERC = E // NS


@functools.lru_cache(maxsize=None)
def _sc_segsum(D):
    @functools.partial(
        pl.kernel,
        out_type=jax.ShapeDtypeStruct((NC * TAB, D), jnp.float32),
        mesh=_mesh,
        compiler_params=_scparams,
        scratch_types=[
            pltpu.VMEM((CHS,), jnp.int32),
            pltpu.VMEM((CHS,), jnp.int32),
            pltpu.VMEM((CHS, D), jnp.float32),
            pltpu.VMEM((RPT, D), jnp.float32),
            pltpu.VMEM_SHARED((TAB, D), jnp.float32),
        ],
    )
    def k(x_hbm, seg_hbm, out_hbm, idx_v, lidx_v, x_v, zbuf_v, table):
        cid = lax.axis_index("c")
        sid = lax.axis_index("s")

        def z2(i, _):
            def z3(j, __):
                zbuf_v[i, pl.ds(j * L, L)] = jnp.zeros((L,), jnp.float32)
                return __
            lax.fori_loop(0, D // L, z3, None)
            return _
        lax.fori_loop(0, RPT, z2, None)
        pltpu.sync_copy(zbuf_v, table.at[pl.ds(sid * RPT, RPT)])
        plsc.subcore_barrier()
        lo = cid * VH

        def body(j, _):
            base = sid * PERC + j * CHS
            pltpu.sync_copy(seg_hbm.at[pl.ds(base, CHS)], idx_v)
            pltpu.sync_copy(x_hbm.at[pl.ds(base, CHS)], x_v)

            def tr(t, __):
                s = idx_v[pl.ds(t * L, L)] - lo
                oob = jnp.logical_or(s < 0, s >= VH)
                lidx_v[pl.ds(t * L, L)] = jnp.where(oob, VH, s)
                return __
            lax.fori_loop(0, CHS // L, tr, None)
            pltpu.sync_copy(x_v, table.at[lidx_v], add=True)
            return _
        lax.fori_loop(0, PERC // CHS, body, None)
        plsc.subcore_barrier()
        pltpu.sync_copy(table.at[pl.ds(sid * RPT, RPT)], zbuf_v)
        pltpu.sync_copy(zbuf_v, out_hbm.at[pl.ds(cid * TAB + sid * RPT, RPT)])

    return k


CHG = 400
NCHG = PERW // CHG


@functools.partial(
    pl.kernel,
    out_type=jax.ShapeDtypeStruct((E, 128), jnp.float32),
    mesh=_mesh,
    compiler_params=_scparams,
    scratch_types=[
        pltpu.VMEM((CHG,), jnp.int32),
        pltpu.VMEM((CHG,), jnp.int32),
        pltpu.VMEM((CHG, 128), jnp.float32),
        pltpu.VMEM((CHG, 128), jnp.float32),
        pltpu.SemaphoreType.DMA,
        pltpu.SemaphoreType.DMA,
    ],
)
def _sc_gather(table_hbm, idx_hbm, out_hbm, idx_a, idx_b, rows_a, rows_b,
               sem0, sem1):
    cid = lax.axis_index("c")
    sid = lax.axis_index("s")
    wid = cid * NS + sid
    base0 = wid * PERW
    idx_v = (idx_a, idx_b)
    rows_v = (rows_a, rows_b)
    sems = (sem0, sem1)
    pltpu.sync_copy(idx_hbm.at[pl.ds(base0, CHG)], idx_a)
    h = pltpu.async_copy(table_hbm.at[idx_a], rows_a, sem0)
    for j in range(NCHG):
        b = j & 1
        h_next = None
        if j + 1 < NCHG:
            nb = (j + 1) & 1
            pltpu.sync_copy(idx_hbm.at[pl.ds(base0 + (j + 1) * CHG, CHG)],
                            idx_v[nb])
            h_next = pltpu.async_copy(table_hbm.at[idx_v[nb]],
                                      rows_v[nb], sems[nb])
        h.wait()
        pltpu.sync_copy(rows_v[b], out_hbm.at[pl.ds(base0 + j * CHG, CHG)])
        h = h_next


def _tc_prep(histp, c1p, c0p, n1p, n0p):
    def body(h_ref, c1_ref, c0_ref, n1_ref, n0_ref,
             hist_ref, urep_ref, c1i_ref, c0i_ref,
             c1r_ref, c0r_ref, n1r_ref, n0r_ref):
        h = h_ref[0] + h_ref[1]
        hist_ref[0, :] = h
        u = jnp.sum((h > 0.0).astype(jnp.int32))
        urep_ref[...] = jnp.full((1, L), u, jnp.int32)
        c1r_ref[0, :] = c1_ref[0] + c1_ref[1]
        c0r_ref[0, :] = c0_ref[0] + c0_ref[1]
        n1r_ref[0, :] = n1_ref[0] + n1_ref[1]
        n0r_ref[0, :] = n0_ref[0] + n0_ref[1]
        c1i_ref[0, :] = 1.0 / jnp.maximum(c1_ref[0] + c1_ref[1], 1.0)
        c0i_ref[0, :] = 1.0 / jnp.maximum(c0_ref[0] + c0_ref[1], 1.0)

    return pl.pallas_call(
        body,
        out_shape=[
            jax.ShapeDtypeStruct((1, NBINS), jnp.float32),
            jax.ShapeDtypeStruct((1, L), jnp.int32),
            jax.ShapeDtypeStruct((1, VP), jnp.float32),
            jax.ShapeDtypeStruct((1, VP), jnp.float32),
            jax.ShapeDtypeStruct((1, VP), jnp.float32),
            jax.ShapeDtypeStruct((1, VP), jnp.float32),
            jax.ShapeDtypeStruct((1, VP), jnp.float32),
            jax.ShapeDtypeStruct((1, VP), jnp.float32),
        ],
    )(histp, c1p, c0p, n1p, n0p)


def _tc_prep2(d1p, d0p):
    def body(a_ref, b_ref, o1_ref, o0_ref):
        o1_ref[0, :] = a_ref[0] + a_ref[1]
        o0_ref[0, :] = b_ref[0] + b_ref[1]

    return pl.pallas_call(
        body,
        out_shape=[
            jax.ShapeDtypeStruct((1, VP), jnp.float32),
            jax.ShapeDtypeStruct((1, VP), jnp.float32),
        ],
    )(d1p, d0p)


def _tc_max(i0r, i1r):
    nb = i0r.shape[0]

    def body(a_ref, b_ref, m0_ref, m1_ref):
        i = pl.program_id(0)

        @pl.when(i == 0)
        def _():
            m0_ref[...] = jnp.zeros((1, 1), jnp.float32)
            m1_ref[...] = jnp.zeros((1, 1), jnp.float32)
        bm0 = jnp.max(a_ref[...]).astype(jnp.float32)
        bm1 = jnp.max(b_ref[...]).astype(jnp.float32)
        m0_ref[...] = jnp.maximum(m0_ref[...], jnp.full((1, 1), bm0, jnp.float32))
        m1_ref[...] = jnp.maximum(m1_ref[...], jnp.full((1, 1), bm1, jnp.float32))

    return pl.pallas_call(
        body,
        grid=(nb,),
        in_specs=[
            pl.BlockSpec((1, 1, i0r.shape[2]), lambda i: (i, 0, 0)),
            pl.BlockSpec((1, 1, i1r.shape[2]), lambda i: (i, 0, 0)),
        ],
        out_specs=[
            pl.BlockSpec((1, 1), lambda i: (0, 0)),
            pl.BlockSpec((1, 1), lambda i: (0, 0)),
        ],
        out_shape=[
            jax.ShapeDtypeStruct((1, 1), jnp.float32),
            jax.ShapeDtypeStruct((1, 1), jnp.float32),
        ],
    )(i0r, i1r)


def _tc_small0(embed, Ar, wrd, Ac, wcd, c1col, c0col, n1col, n0col, d1col, d0col):
    blk = 2048

    def body(e_ref, ar_ref, wrd_ref, ac_ref, wcd_ref,
             c1_ref, c0_ref, n1_ref, n0_ref, d1_ref, d0_ref,
             rq_ref, cq_ref, sn_ref, sd_ref):
        i = pl.program_id(0)
        e0 = e_ref[0:1, :]
        de = e_ref[1:2, :] - e0
        p0r = jnp.dot(e0, ar_ref[...].T, preferred_element_type=jnp.float32)
        pdr = jnp.dot(de, ar_ref[...].T, preferred_element_type=jnp.float32)
        p0c = jnp.dot(e0, ac_ref[...].T, preferred_element_type=jnp.float32)
        pdc = jnp.dot(de, ac_ref[...].T, preferred_element_type=jnp.float32)
        c1 = c1_ref[...]
        c0 = c0_ref[...]
        n1 = n1_ref[...]
        n0 = n0_ref[...]
        d1 = d1_ref[...]
        d0 = d0_ref[...]
        rq_ref[...] = (c1 * p0r + n1 * pdr + d1 * wrd_ref[...]) / jnp.maximum(c1, 1.0)
        cq_ref[...] = (c0 * p0c + n0 * pdc + d0 * wcd_ref[...]) / jnp.maximum(c0, 1.0)

        @pl.when(i == 0)
        def _():
            sn_ref[...] = jnp.zeros((1, 1), jnp.float32)
            sd_ref[...] = jnp.zeros((1, 1), jnp.float32)
        sn_ref[...] += jnp.sum(n1, keepdims=True).reshape(1, 1)
        sd_ref[...] += jnp.sum(d1, keepdims=True).reshape(1, 1)

    return pl.pallas_call(
        body,
        grid=(VP // blk,),
        in_specs=[
            pl.BlockSpec((2, 128), lambda i: (0, 0)),
            pl.BlockSpec((128, 128), lambda i: (0, 0)),
            pl.BlockSpec((1, 128), lambda i: (0, 0)),
            pl.BlockSpec((128, 128), lambda i: (0, 0)),
            pl.BlockSpec((1, 128), lambda i: (0, 0)),
            pl.BlockSpec((blk, 1), lambda i: (i, 0)),
            pl.BlockSpec((blk, 1), lambda i: (i, 0)),
            pl.BlockSpec((blk, 1), lambda i: (i, 0)),
            pl.BlockSpec((blk, 1), lambda i: (i, 0)),
            pl.BlockSpec((blk, 1), lambda i: (i, 0)),
            pl.BlockSpec((blk, 1), lambda i: (i, 0)),
        ],
        out_specs=[
            pl.BlockSpec((blk, 128), lambda i: (i, 0)),
            pl.BlockSpec((blk, 128), lambda i: (i, 0)),
            pl.BlockSpec((1, 1), lambda i: (0, 0)),
            pl.BlockSpec((1, 1), lambda i: (0, 0)),
        ],
        out_shape=[
            jax.ShapeDtypeStruct((VP, 128), jnp.float32),
            jax.ShapeDtypeStruct((VP, 128), jnp.float32),
            jax.ShapeDtypeStruct((1, 1), jnp.float32),
            jax.ShapeDtypeStruct((1, 1), jnp.float32),
        ],
    )(embed, Ar, wrd, Ac, wcd, c1col, c0col, n1col, n0col, d1col, d0col)


def _tc_big0(flat2, deg2, G1, G0, embed, Av, wvd, Am, wmd, b2, Sn, Sd):
    blk = 2000

    def body(f_ref, d_ref, g1_ref, g0_ref, e_ref, av_ref, wvd_ref,
             am_ref, wmd_ref, b_ref, sn_ref, sd_ref, o_ref):
        e0 = e_ref[0:1, :]
        de = e_ref[1:2, :] - e0
        q0 = jnp.dot(e0, av_ref[...].T, preferred_element_type=jnp.float32)
        qd = jnp.dot(de, av_ref[...].T, preferred_element_type=jnp.float32)
        p0m = jnp.dot(e0, am_ref[...].T, preferred_element_type=jnp.float32)
        pdm = jnp.dot(de, am_ref[...].T, preferred_element_type=jnp.float32)
        mvec = (p0m + (sn_ref[...] * (1.0 / E)) * pdm
                + (sd_ref[...] * (1.0 / E)) * wmd_ref[...] + b_ref[...])
        pre = (q0 + f_ref[...] * qd + d_ref[...] * wvd_ref[...]
               + g1_ref[...] + g0_ref[...] + mvec)
        o_ref[...] = jnp.where(pre >= 0.0, pre, 0.01 * pre)

    return pl.pallas_call(
        body,
        grid=(E // blk,),
        in_specs=[
            pl.BlockSpec((blk, 1), lambda i: (i, 0)),
            pl.BlockSpec((blk, 1), lambda i: (i, 0)),
            pl.BlockSpec((blk, 128), lambda i: (i, 0)),
            pl.BlockSpec((blk, 128), lambda i: (i, 0)),
            pl.BlockSpec((2, 128), lambda i: (0, 0)),
            pl.BlockSpec((128, 128), lambda i: (0, 0)),
            pl.BlockSpec((1, 128), lambda i: (0, 0)),
            pl.BlockSpec((128, 128), lambda i: (0, 0)),
            pl.BlockSpec((1, 128), lambda i: (0, 0)),
            pl.BlockSpec((1, 128), lambda i: (0, 0)),
            pl.BlockSpec((1, 1), lambda i: (0, 0)),
            pl.BlockSpec((1, 1), lambda i: (0, 0)),
        ],
        out_specs=pl.BlockSpec((blk, 128), lambda i: (i, 0)),
        out_shape=jax.ShapeDtypeStruct((E, 128), jnp.float32),
    )(flat2, deg2, G1, G0, embed, Av, wvd, Am, wmd, b2, Sn, Sd)


def _tc_small(Rp, Cp, c1col, c0col, Wr, Wc):
    D = Rp.shape[1]
    blk = 2048

    def body(r_ref, c_ref, c1_ref, c0_ref, wr_ref, wc_ref,
             rq_ref, cq_ref, cs_ref):
        i = pl.program_id(0)
        Rs = r_ref[...]
        Cs = c_ref[...]
        rq_ref[...] = jnp.dot(Rs, wr_ref[...].T, preferred_element_type=jnp.float32) * c1_ref[...]
        cq_ref[...] = jnp.dot(Cs, wc_ref[...].T, preferred_element_type=jnp.float32) * c0_ref[...]

        @pl.when(i == 0)
        def _():
            cs_ref[...] = jnp.zeros((1, D), jnp.float32)
        cs_ref[...] += jnp.sum(Rs, axis=0, keepdims=True)

    return pl.pallas_call(
        body,
        grid=(VP // blk,),
        in_specs=[
            pl.BlockSpec((blk, D), lambda i: (i, 0)),
            pl.BlockSpec((blk, D), lambda i: (i, 0)),
            pl.BlockSpec((blk, 1), lambda i: (i, 0)),
            pl.BlockSpec((blk, 1), lambda i: (i, 0)),
            pl.BlockSpec(Wr.shape, lambda i: (0, 0)),
            pl.BlockSpec(Wc.shape, lambda i: (0, 0)),
        ],
        out_specs=[
            pl.BlockSpec((blk, 128), lambda i: (i, 0)),
            pl.BlockSpec((blk, 128), lambda i: (i, 0)),
            pl.BlockSpec((1, D), lambda i: (0, 0)),
        ],
        out_shape=[
            jax.ShapeDtypeStruct((VP, 128), jnp.float32),
            jax.ShapeDtypeStruct((VP, 128), jnp.float32),
            jax.ShapeDtypeStruct((1, D), jnp.float32),
        ],
    )(Rp, Cp, c1col, c0col, Wr, Wc)


def _tc_big(v, G1, G0, Wv, colsum, Wm, b2):
    D = v.shape[1]
    blk = 2000

    def body(v_ref, g1_ref, g0_ref, wv_ref, cs_ref, wm_ref, b_ref, o_ref):
        mvec = jnp.dot(cs_ref[...] * (1.0 / E), wm_ref[...].T,
                       preferred_element_type=jnp.float32) + b_ref[...]
        pre = (jnp.dot(v_ref[...], wv_ref[...].T, preferred_element_type=jnp.float32)
               + g1_ref[...] + g0_ref[...] + mvec)
        o_ref[...] = jnp.where(pre >= 0.0, pre, 0.01 * pre)

    return pl.pallas_call(
        body,
        grid=(E // blk,),
        in_specs=[
            pl.BlockSpec((blk, D), lambda i: (i, 0)),
            pl.BlockSpec((blk, 128), lambda i: (i, 0)),
            pl.BlockSpec((blk, 128), lambda i: (i, 0)),
            pl.BlockSpec(Wv.shape, lambda i: (0, 0)),
            pl.BlockSpec((1, D), lambda i: (0, 0)),
            pl.BlockSpec(Wm.shape, lambda i: (0, 0)),
            pl.BlockSpec((1, 128), lambda i: (0, 0)),
        ],
        out_specs=pl.BlockSpec((blk, 128), lambda i: (i, 0)),
        out_shape=jax.ShapeDtypeStruct((E, 128), jnp.float32),
    )(v, G1, G0, Wv, colsum, Wm, b2)


def _tc_final(emp, W1, b1, W2, b2, W3p, b3p):
    blk = 2048

    def body(e_ref, w1_ref, b1_ref, w2_ref, b2_ref, w3_ref, b3_ref,
             lg_ref, cs_ref):
        i = pl.program_id(0)
        em = e_ref[...]
        h = jnp.maximum(jnp.dot(em, w1_ref[...].T, preferred_element_type=jnp.float32) + b1_ref[...], 0.0)
        h = jnp.maximum(jnp.dot(h, w2_ref[...].T, preferred_element_type=jnp.float32) + b2_ref[...], 0.0)
        lg_ref[...] = jnp.dot(h, w3_ref[...].T, preferred_element_type=jnp.float32) + b3_ref[...]

        @pl.when(i == 0)
        def _():
            cs_ref[...] = jnp.zeros((1, 128), jnp.float32)
        cs_ref[...] += jnp.sum(em, axis=0, keepdims=True)

    return pl.pallas_call(
        body,
        grid=(VP // blk,),
        in_specs=[
            pl.BlockSpec((blk, 128), lambda i: (i, 0)),
            pl.BlockSpec(W1.shape, lambda i: (0, 0)),
            pl.BlockSpec((1, 128), lambda i: (0, 0)),
            pl.BlockSpec(W2.shape, lambda i: (0, 0)),
            pl.BlockSpec((1, 128), lambda i: (0, 0)),
            pl.BlockSpec(W3p.shape, lambda i: (0, 0)),
            pl.BlockSpec((1, 128), lambda i: (0, 0)),
        ],
        out_specs=[
            pl.BlockSpec((blk, 128), lambda i: (i, 0)),
            pl.BlockSpec((1, 128), lambda i: (0, 0)),
        ],
        out_shape=[
            jax.ShapeDtypeStruct((VP, 128), jnp.float32),
            jax.ShapeDtypeStruct((1, 128), jnp.float32),
        ],
    )(emp, W1, b1, W2, b2, W3p, b3p)


def _tc_value(emcol, sfp, W1p, b1, W2, b2, W3p, b3p):
    def body(ec_ref, sf_ref, w1_ref, b1_ref, w2_ref, b2_ref, w3_ref, b3_ref, o_ref):
        x = jnp.concatenate([ec_ref[...] * (1.0 / V), sf_ref[...]], axis=1)
        h = jnp.maximum(jnp.dot(x, w1_ref[...].T, preferred_element_type=jnp.float32) + b1_ref[...], 0.0)
        h = jnp.maximum(jnp.dot(h, w2_ref[...].T, preferred_element_type=jnp.float32) + b2_ref[...], 0.0)
        o_ref[...] = jnp.dot(h, w3_ref[...].T, preferred_element_type=jnp.float32) + b3_ref[...]

    return pl.pallas_call(
        body,
        out_shape=jax.ShapeDtypeStruct((1, 128), jnp.float32),
    )(emcol, sfp, W1p, b1, W2, b2, W3p, b3p)


def kernel(indices, values, embed,
           ex0_W, ex0_b, ex1_W, ex1_b, ex2_W, ex2_b, ex3_W, ex3_b, ex4_W, ex4_b,
           ex5_W, ex5_b, ex6_W, ex6_b, ex7_W, ex7_b, ex8_W, ex8_b,
           cl_W1, cl_b1, cl_W2, cl_b2, cl_W3, cl_b3,
           vl_W1, vl_b1, vl_W2, vl_b2, vl_W3, vl_b3):
    exW = [ex0_W, ex1_W, ex2_W, ex3_W, ex4_W, ex5_W, ex6_W, ex7_W, ex8_W]
    exb = [ex0_b, ex1_b, ex2_b, ex3_b, ex4_b, ex5_b, ex6_b, ex7_b, ex8_b]

    ind0 = indices[0].astype(jnp.int32)
    ind1 = indices[1].astype(jnp.int32)
    flat = values[:, 0]

    histp, c1p, c0p, n1p, n0p = _sc_stats(ind0, ind1, flat)
    hist2, urep2, c1i, c0i, c1r, c0r, n1r, n0r = _tc_prep(histp, c1p, c0p, n1p, n0p)
    compact = _sc_compact(hist2.reshape(NBINS))
    degree, dg1p, dg0p = _sc_degree(ind1, ind0, compact, urep2.reshape(L))
    dg1, dg0 = _tc_prep2(dg1p, dg0p)
    m0, m1 = _tc_max(ind0.reshape(E // 2000, 1, 2000), ind1.reshape(E // 2000, 1, 2000))

    c1col = c1i.reshape(VP, 1)
    c0col = c0i.reshape(VP, 1)

    W0 = exW[0]
    Wv0, Wr0, Wc0, Wm0 = W0[:, :129], W0[:, 129:258], W0[:, 258:387], W0[:, 387:516]
    Av, wvd = Wv0[:, :128], Wv0[:, 128].reshape(1, 128)
    Ar, wrd = Wr0[:, :128], Wr0[:, 128].reshape(1, 128)
    Ac, wcd = Wc0[:, :128], Wc0[:, 128].reshape(1, 128)
    Am, wmd = Wm0[:, :128], Wm0[:, 128].reshape(1, 128)
    Rq, Cq, Sn, Sd = _tc_small0(
        embed, Ar, wrd, Ac, wcd,
        c1r.reshape(VP, 1), c0r.reshape(VP, 1),
        n1r.reshape(VP, 1), n0r.reshape(VP, 1),
        dg1.reshape(VP, 1), dg0.reshape(VP, 1))
    G1 = _sc_gather(Rq, ind1)
    G0 = _sc_gather(Cq, ind0)
    v = _tc_big0(flat.reshape(E, 1), degree.reshape(E, 1), G1, G0, embed,
                 Av, wvd, Am, wmd, exb[0].reshape(1, 128), Sn, Sd)

    for i in range(1, 9):
        W = exW[i]
        Wv, Wr, Wc, Wm = W[:, :128], W[:, 128:256], W[:, 256:384], W[:, 384:512]
        seg = _sc_segsum(128)
        Rp = seg(v, ind1)
        Cp = seg(v, ind0)
        Rsum = jnp.concatenate([Rp[:VH], Rp[TAB:TAB + VH]], axis=0)
        Csum = jnp.concatenate([Cp[:VH], Cp[TAB:TAB + VH]], axis=0)
        Rq, Cq, colsum = _tc_small(Rsum, Csum, c1col, c0col, Wr, Wc)
        G1 = _sc_gather(Rq, ind1)
        G0 = _sc_gather(Cq, ind0)
        v = _tc_big(v, G1, G0, Wv, colsum, Wm, exb[i].reshape(1, 128))

    emp = _sc_segsum(128)(v, ind1)
    em = jnp.concatenate([emp[:VH], emp[TAB:TAB + VH]], axis=0)
    W3p = jnp.zeros((128, 128), jnp.float32).at[:2, :].set(cl_W3)
    b3p = jnp.zeros((1, 128), jnp.float32).at[0, :2].set(cl_b3)
    logits, emcol = _tc_final(em, cl_W1, cl_b1.reshape(1, 128),
                              cl_W2, cl_b2.reshape(1, 128), W3p, b3p)

    sfp = jnp.zeros((1, 16), jnp.float32)
    sfp = sfp.at[0, 0].set(float(E) / 100.0)
    sfp = sfp.at[0, 1].set(m0[0, 0] / 100.0)
    sfp = sfp.at[0, 2].set(m1[0, 0] / 100.0)
    vW1p = jnp.zeros((128, 144), jnp.float32)
    vW1p = vW1p.at[:, :128].set(vl_W1[:, :128])
    vW1p = vW1p.at[:, 128:131].set(vl_W1[:, 128:131])
    vW3p = jnp.zeros((128, 128), jnp.float32).at[:1, :].set(vl_W3)
    vb3p = jnp.zeros((1, 128), jnp.float32).at[0, :1].set(vl_b3)
    val = _tc_value(emcol, sfp, vW1p, vl_b1.reshape(1, 128),
                    vl_W2, vl_b2.reshape(1, 128), vW3p, vb3p)

    counts_out = logits[:V, :2].reshape(-1)
    return jnp.concatenate([counts_out, val[0, :1]])


# speedup vs baseline: 2.6688x; 1.2094x over previous
import functools

import jax
import jax.numpy as jnp
from jax import lax
from jax.experimental import pallas as pl
from jax.experimental.pallas import tpu as pltpu, tpu_sc as plsc

E = 320000
V = 10000
VP = 10240
NBINS = 20480
NC, NS, L = 2, 16, 16
NW = NC * NS
PERW = E // NW
CH = 200
NCH = PERW // CH

_mesh = plsc.VectorSubcoreMesh(core_axis_name="c", subcore_axis_name="s")
_scparams = pltpu.CompilerParams(needs_layout_passes=False)


def _zero_vmem(ref, n):
    def z(i, _):
        ref[pl.ds(i * L, L)] = jnp.zeros((L,), jnp.float32)
        return _
    lax.fori_loop(0, n // L, z, None)


@functools.partial(
    pl.kernel,
    out_type=[
        jax.ShapeDtypeStruct((NC, NBINS), jnp.float32),
        jax.ShapeDtypeStruct((NC, VP), jnp.float32),
        jax.ShapeDtypeStruct((NC, VP), jnp.float32),
        jax.ShapeDtypeStruct((NC, VP), jnp.float32),
        jax.ShapeDtypeStruct((NC, VP), jnp.float32),
    ],
    mesh=_mesh,
    compiler_params=_scparams,
    scratch_types=[
        pltpu.VMEM((PERW,), jnp.int32),
        pltpu.VMEM((PERW,), jnp.int32),
        pltpu.VMEM((PERW,), jnp.float32),
        pltpu.VMEM((NBINS,), jnp.float32),
        pltpu.VMEM((VP,), jnp.float32),
        pltpu.VMEM((VP,), jnp.float32),
        pltpu.VMEM((VP,), jnp.float32),
        pltpu.VMEM((VP,), jnp.float32),
        pltpu.VMEM((NBINS // NS,), jnp.float32),
        pltpu.VMEM((NBINS // NS,), jnp.float32),
        pltpu.VMEM_SHARED((NS, NBINS), jnp.float32),
    ],
)
def _sc_stats(i0_hbm, i1_hbm, f_hbm, hist_hbm, c1_hbm, c0_hbm, n1_hbm, n0_hbm,
              i0_v, i1_v, f_v, hist_v, c1_v, c0_v, n1_v, n0_v, acc_v, tmp_v,
              sh_h):
    cid = lax.axis_index("c")
    sid = lax.axis_index("s")
    wid = cid * NS + sid
    _zero_vmem(hist_v, NBINS)
    _zero_vmem(c1_v, VP)
    _zero_vmem(c0_v, VP)
    _zero_vmem(n1_v, VP)
    _zero_vmem(n0_v, VP)
    base = wid * PERW
    pltpu.sync_copy(i0_hbm.at[pl.ds(base, PERW)], i0_v)
    pltpu.sync_copy(i1_hbm.at[pl.ds(base, PERW)], i1_v)
    pltpu.sync_copy(f_hbm.at[pl.ds(base, PERW)], f_v)
    ones = jnp.ones((L,), jnp.float32)

    def body(j, _):
        i1 = i1_v[pl.ds(j * L, L)]
        i0 = i0_v[pl.ds(j * L, L)]
        f = f_v[pl.ds(j * L, L)]
        binv = jnp.where(f == 0.0, -i1, i1) + 9999
        plsc.addupdate_scatter(hist_v, [binv], ones)
        plsc.addupdate_scatter(c1_v, [i1], ones)
        plsc.addupdate_scatter(c0_v, [i0], ones)
        plsc.addupdate_scatter(n1_v, [i1], f)
        plsc.addupdate_scatter(n0_v, [i0], f)
        return _

    lax.fori_loop(0, PERW // L, body, None)

    def reduce_out(local_v, shared, out_ref, size):
        pltpu.sync_copy(local_v, shared.at[sid, pl.ds(0, size)])
        plsc.subcore_barrier()
        sl = size // NS
        rbase = sid * sl
        pltpu.sync_copy(shared.at[0, pl.ds(rbase, sl)], acc_v.at[pl.ds(0, sl)])

        def red(k, _):
            pltpu.sync_copy(shared.at[k, pl.ds(rbase, sl)], tmp_v.at[pl.ds(0, sl)])

            def addv(i, __):
                acc_v[pl.ds(i * L, L)] = acc_v[pl.ds(i * L, L)] + tmp_v[pl.ds(i * L, L)]
                return __
            lax.fori_loop(0, sl // L, addv, None)
            return _
        lax.fori_loop(1, NS, red, None)
        pltpu.sync_copy(acc_v.at[pl.ds(0, sl)], out_ref.at[cid, pl.ds(rbase, sl)])
        plsc.subcore_barrier()

    reduce_out(hist_v, sh_h, hist_hbm, NBINS)
    reduce_out(c1_v, sh_h, c1_hbm, VP)
    reduce_out(c0_v, sh_h, c0_hbm, VP)
    reduce_out(n1_v, sh_h, n1_hbm, VP)
    reduce_out(n0_v, sh_h, n0_hbm, VP)


@functools.partial(
    pl.kernel,
    out_type=jax.ShapeDtypeStruct((NBINS,), jnp.float32),
    mesh=_mesh,
    compiler_params=_scparams,
    scratch_types=[
        pltpu.VMEM((NBINS,), jnp.float32),
        pltpu.VMEM((NBINS,), jnp.float32),
    ],
)
def _sc_compact(hist_hbm, out_hbm, hist_v, comp_v):
    cid = lax.axis_index("c")
    sid = lax.axis_index("s")

    @pl.when(jnp.logical_and(cid == 0, sid == 0))
    def _():
        pltpu.sync_copy(hist_hbm, hist_v)
        _zero_vmem(comp_v, NBINS)

        def body(j, carry):
            v = hist_v[pl.ds(j * L, L)]
            mask = v > 0.0
            mi = mask.astype(jnp.int32)
            cs = plsc.cumsum(mi)
            pos = jnp.maximum(carry + cs - 1, 0)
            plsc.store_scatter(comp_v, [pos], v, mask=mask)
            return carry + jnp.sum(mi)

        lax.fori_loop(0, NBINS // L, body, jnp.int32(0))
        pltpu.sync_copy(comp_v, out_hbm)


@functools.partial(
    pl.kernel,
    out_type=[
        jax.ShapeDtypeStruct((E,), jnp.float32),
        jax.ShapeDtypeStruct((NC, VP), jnp.float32),
        jax.ShapeDtypeStruct((NC, VP), jnp.float32),
    ],
    mesh=_mesh,
    compiler_params=_scparams,
    scratch_types=[
        pltpu.VMEM((NBINS,), jnp.float32),
        pltpu.VMEM((L,), jnp.int32),
        pltpu.VMEM((PERW,), jnp.int32),
        pltpu.VMEM((PERW,), jnp.int32),
        pltpu.VMEM((PERW,), jnp.float32),
        pltpu.VMEM((VP,), jnp.float32),
        pltpu.VMEM((VP,), jnp.float32),
        pltpu.VMEM((VP // NS,), jnp.float32),
        pltpu.VMEM((VP // NS,), jnp.float32),
        pltpu.VMEM_SHARED((NS, VP), jnp.float32),
    ],
)
def _sc_degree(i1_hbm, i0_hbm, comp_hbm, urep_hbm,
               deg_hbm, dg1_hbm, dg0_hbm,
               comp_v, u_v, i1_v, i0_v, deg_v, dg1_v, dg0_v, acc_v, tmp_v, sh):
    cid = lax.axis_index("c")
    sid = lax.axis_index("s")
    wid = cid * NS + sid
    base = wid * PERW
    pltpu.sync_copy(comp_hbm, comp_v)
    pltpu.sync_copy(urep_hbm, u_v)
    pltpu.sync_copy(i1_hbm.at[pl.ds(base, PERW)], i1_v)
    pltpu.sync_copy(i0_hbm.at[pl.ds(base, PERW)], i0_v)
    _zero_vmem(dg1_v, VP)
    _zero_vmem(dg0_v, VP)

    def body(j, _):
        i1 = i1_v[pl.ds(j * L, L)]
        i0 = i0_v[pl.ds(j * L, L)]
        um = u_v[...]
        ic = jnp.maximum(jnp.minimum(i1, um - 1), 0)
        d = plsc.load_gather(comp_v, [ic])
        deg_v[pl.ds(j * L, L)] = d
        plsc.addupdate_scatter(dg1_v, [i1], d)
        plsc.addupdate_scatter(dg0_v, [i0], d)
        return _

    lax.fori_loop(0, PERW // L, body, None)
    pltpu.sync_copy(deg_v, deg_hbm.at[pl.ds(base, PERW)])

    def reduce_out(local_v, out_ref):
        pltpu.sync_copy(local_v, sh.at[sid])
        plsc.subcore_barrier()
        sl = VP // NS
        rbase = sid * sl
        pltpu.sync_copy(sh.at[0, pl.ds(rbase, sl)], acc_v)

        def red(k, _):
            pltpu.sync_copy(sh.at[k, pl.ds(rbase, sl)], tmp_v)

            def addv(i, __):
                acc_v[pl.ds(i * L, L)] = acc_v[pl.ds(i * L, L)] + tmp_v[pl.ds(i * L, L)]
                return __
            lax.fori_loop(0, sl // L, addv, None)
            return _
        lax.fori_loop(1, NS, red, None)
        pltpu.sync_copy(acc_v, out_ref.at[cid, pl.ds(rbase, sl)])
        plsc.subcore_barrier()

    reduce_out(dg1_v, dg1_hbm)
    reduce_out(dg0_v, dg0_hbm)


VH = VP // 2
TAB = VH + 128
RPT = TAB // NS
CHS = 160
PERC = E // NS


@functools.lru_cache(maxsize=None)
def _sc_segsum(D):
    @functools.partial(
        pl.kernel,
        out_type=jax.ShapeDtypeStruct((NC * TAB, D), jnp.float32),
        mesh=_mesh,
        compiler_params=_scparams,
        scratch_types=[
            pltpu.VMEM((CHS,), jnp.int32),
            pltpu.VMEM((CHS,), jnp.int32),
            pltpu.VMEM((CHS,), jnp.int32),
            pltpu.VMEM((CHS,), jnp.int32),
            pltpu.VMEM((CHS, D), jnp.float32),
            pltpu.VMEM((CHS, D), jnp.float32),
            pltpu.VMEM((RPT, D), jnp.float32),
            pltpu.VMEM_SHARED((TAB, D), jnp.float32),
            pltpu.SemaphoreType.DMA,
            pltpu.SemaphoreType.DMA,
        ],
    )
    def k(x_hbm, seg_hbm, out_hbm, seg_a, seg_b, lidx_a, lidx_b,
          x_a, x_b, zbuf_v, table, sem_a, sem_b):
        cid = lax.axis_index("c")
        sid = lax.axis_index("s")

        def z2(i, _):
            def z3(j, __):
                zbuf_v[i, pl.ds(j * L, L)] = jnp.zeros((L,), jnp.float32)
                return __
            lax.fori_loop(0, D // L, z3, None)
            return _
        lax.fori_loop(0, RPT, z2, None)
        pltpu.sync_copy(zbuf_v, table.at[pl.ds(sid * RPT, RPT)])
        plsc.subcore_barrier()
        lo = cid * VH
        nchs = PERC // CHS

        def issue(ch, segbuf, xbuf, sem):
            base = sid * PERC + ch * CHS
            pltpu.async_copy(seg_hbm.at[pl.ds(base, CHS)], segbuf, sem)
            pltpu.async_copy(x_hbm.at[pl.ds(base, CHS)], xbuf, sem)

        def wait(ch, segbuf, xbuf, sem):
            base = sid * PERC + ch * CHS
            pltpu.make_async_copy(seg_hbm.at[pl.ds(base, CHS)], segbuf, sem).wait()
            pltpu.make_async_copy(x_hbm.at[pl.ds(base, CHS)], xbuf, sem).wait()

        def scatter(segbuf, lidxbuf, xbuf):
            def tr(t, __):
                s = segbuf[pl.ds(t * L, L)] - lo
                oob = jnp.logical_or(s < 0, s >= VH)
                lidxbuf[pl.ds(t * L, L)] = jnp.where(oob, VH, s)
                return __
            lax.fori_loop(0, CHS // L, tr, None)
            pltpu.sync_copy(xbuf, table.at[lidxbuf], add=True)

        issue(0, seg_a, x_a, sem_a)

        def body(i, _):
            ch0 = 2 * i
            issue(ch0 + 1, seg_b, x_b, sem_b)
            wait(ch0, seg_a, x_a, sem_a)
            scatter(seg_a, lidx_a, x_a)
            issue(ch0 + 2, seg_a, x_a, sem_a)
            wait(ch0 + 1, seg_b, x_b, sem_b)
            scatter(seg_b, lidx_b, x_b)
            return _
        lax.fori_loop(0, (nchs - 1) // 2, body, None)
        wait(nchs - 1, seg_a, x_a, sem_a)
        scatter(seg_a, lidx_a, x_a)
        plsc.subcore_barrier()
        pltpu.sync_copy(table.at[pl.ds(sid * RPT, RPT)], zbuf_v)
        pltpu.sync_copy(zbuf_v, out_hbm.at[pl.ds(cid * TAB + sid * RPT, RPT)])

    return k


CHG = 400
NCHG = PERW // CHG


@functools.partial(
    pl.kernel,
    out_type=jax.ShapeDtypeStruct((E, 128), jnp.float32),
    mesh=_mesh,
    compiler_params=_scparams,
    scratch_types=[
        pltpu.VMEM((CHG,), jnp.int32),
        pltpu.VMEM((CHG,), jnp.int32),
        pltpu.VMEM((CHG, 128), jnp.float32),
        pltpu.VMEM((CHG, 128), jnp.float32),
        pltpu.SemaphoreType.DMA,
        pltpu.SemaphoreType.DMA,
    ],
)
def _sc_gather(table_hbm, idx_hbm, out_hbm, idx_a, idx_b, rows_a, rows_b,
               sem0, sem1):
    cid = lax.axis_index("c")
    sid = lax.axis_index("s")
    wid = cid * NS + sid
    base0 = wid * PERW
    idx_v = (idx_a, idx_b)
    rows_v = (rows_a, rows_b)
    sems = (sem0, sem1)
    pltpu.sync_copy(idx_hbm.at[pl.ds(base0, CHG)], idx_a)
    h = pltpu.async_copy(table_hbm.at[idx_a], rows_a, sem0)
    for j in range(NCHG):
        b = j & 1
        h_next = None
        if j + 1 < NCHG:
            nb = (j + 1) & 1
            pltpu.sync_copy(idx_hbm.at[pl.ds(base0 + (j + 1) * CHG, CHG)],
                            idx_v[nb])
            h_next = pltpu.async_copy(table_hbm.at[idx_v[nb]],
                                      rows_v[nb], sems[nb])
        h.wait()
        pltpu.sync_copy(rows_v[b], out_hbm.at[pl.ds(base0 + j * CHG, CHG)])
        h = h_next


def _tc_prep(histp, c1p, c0p, n1p, n0p):
    def body(h_ref, c1_ref, c0_ref, n1_ref, n0_ref,
             hist_ref, urep_ref, c1i_ref, c0i_ref,
             c1r_ref, c0r_ref, n1r_ref, n0r_ref):
        h = h_ref[0] + h_ref[1]
        hist_ref[0, :] = h
        u = jnp.sum((h > 0.0).astype(jnp.int32))
        urep_ref[...] = jnp.full((1, L), u, jnp.int32)
        c1r_ref[0, :] = c1_ref[0] + c1_ref[1]
        c0r_ref[0, :] = c0_ref[0] + c0_ref[1]
        n1r_ref[0, :] = n1_ref[0] + n1_ref[1]
        n0r_ref[0, :] = n0_ref[0] + n0_ref[1]
        c1i_ref[0, :] = 1.0 / jnp.maximum(c1_ref[0] + c1_ref[1], 1.0)
        c0i_ref[0, :] = 1.0 / jnp.maximum(c0_ref[0] + c0_ref[1], 1.0)

    return pl.pallas_call(
        body,
        out_shape=[
            jax.ShapeDtypeStruct((1, NBINS), jnp.float32),
            jax.ShapeDtypeStruct((1, L), jnp.int32),
            jax.ShapeDtypeStruct((1, VP), jnp.float32),
            jax.ShapeDtypeStruct((1, VP), jnp.float32),
            jax.ShapeDtypeStruct((1, VP), jnp.float32),
            jax.ShapeDtypeStruct((1, VP), jnp.float32),
            jax.ShapeDtypeStruct((1, VP), jnp.float32),
            jax.ShapeDtypeStruct((1, VP), jnp.float32),
        ],
    )(histp, c1p, c0p, n1p, n0p)


def _tc_prep2(d1p, d0p):
    def body(a_ref, b_ref, o1_ref, o0_ref):
        o1_ref[0, :] = a_ref[0] + a_ref[1]
        o0_ref[0, :] = b_ref[0] + b_ref[1]

    return pl.pallas_call(
        body,
        out_shape=[
            jax.ShapeDtypeStruct((1, VP), jnp.float32),
            jax.ShapeDtypeStruct((1, VP), jnp.float32),
        ],
    )(d1p, d0p)


def _tc_max(i0r, i1r):
    nb = i0r.shape[0]

    def body(a_ref, b_ref, m0_ref, m1_ref):
        i = pl.program_id(0)

        @pl.when(i == 0)
        def _():
            m0_ref[...] = jnp.zeros((1, 1), jnp.float32)
            m1_ref[...] = jnp.zeros((1, 1), jnp.float32)
        bm0 = jnp.max(a_ref[...]).astype(jnp.float32)
        bm1 = jnp.max(b_ref[...]).astype(jnp.float32)
        m0_ref[...] = jnp.maximum(m0_ref[...], jnp.full((1, 1), bm0, jnp.float32))
        m1_ref[...] = jnp.maximum(m1_ref[...], jnp.full((1, 1), bm1, jnp.float32))

    return pl.pallas_call(
        body,
        grid=(nb,),
        in_specs=[
            pl.BlockSpec((1, 1, i0r.shape[2]), lambda i: (i, 0, 0)),
            pl.BlockSpec((1, 1, i1r.shape[2]), lambda i: (i, 0, 0)),
        ],
        out_specs=[
            pl.BlockSpec((1, 1), lambda i: (0, 0)),
            pl.BlockSpec((1, 1), lambda i: (0, 0)),
        ],
        out_shape=[
            jax.ShapeDtypeStruct((1, 1), jnp.float32),
            jax.ShapeDtypeStruct((1, 1), jnp.float32),
        ],
    )(i0r, i1r)


def _tc_small0(embed, Ar, wrd, Ac, wcd, c1col, c0col, n1col, n0col, d1col, d0col):
    blk = 2048

    def body(e_ref, ar_ref, wrd_ref, ac_ref, wcd_ref,
             c1_ref, c0_ref, n1_ref, n0_ref, d1_ref, d0_ref,
             rq_ref, cq_ref, sn_ref, sd_ref):
        i = pl.program_id(0)
        e0 = e_ref[0:1, :]
        de = e_ref[1:2, :] - e0
        p0r = jnp.dot(e0, ar_ref[...].T, preferred_element_type=jnp.float32)
        pdr = jnp.dot(de, ar_ref[...].T, preferred_element_type=jnp.float32)
        p0c = jnp.dot(e0, ac_ref[...].T, preferred_element_type=jnp.float32)
        pdc = jnp.dot(de, ac_ref[...].T, preferred_element_type=jnp.float32)
        c1 = c1_ref[...]
        c0 = c0_ref[...]
        n1 = n1_ref[...]
        n0 = n0_ref[...]
        d1 = d1_ref[...]
        d0 = d0_ref[...]
        rq_ref[...] = (c1 * p0r + n1 * pdr + d1 * wrd_ref[...]) / jnp.maximum(c1, 1.0)
        cq_ref[...] = (c0 * p0c + n0 * pdc + d0 * wcd_ref[...]) / jnp.maximum(c0, 1.0)

        @pl.when(i == 0)
        def _():
            sn_ref[...] = jnp.zeros((1, 1), jnp.float32)
            sd_ref[...] = jnp.zeros((1, 1), jnp.float32)
        sn_ref[...] += jnp.sum(n1, keepdims=True).reshape(1, 1)
        sd_ref[...] += jnp.sum(d1, keepdims=True).reshape(1, 1)

    return pl.pallas_call(
        body,
        grid=(VP // blk,),
        in_specs=[
            pl.BlockSpec((2, 128), lambda i: (0, 0)),
            pl.BlockSpec((128, 128), lambda i: (0, 0)),
            pl.BlockSpec((1, 128), lambda i: (0, 0)),
            pl.BlockSpec((128, 128), lambda i: (0, 0)),
            pl.BlockSpec((1, 128), lambda i: (0, 0)),
            pl.BlockSpec((blk, 1), lambda i: (i, 0)),
            pl.BlockSpec((blk, 1), lambda i: (i, 0)),
            pl.BlockSpec((blk, 1), lambda i: (i, 0)),
            pl.BlockSpec((blk, 1), lambda i: (i, 0)),
            pl.BlockSpec((blk, 1), lambda i: (i, 0)),
            pl.BlockSpec((blk, 1), lambda i: (i, 0)),
        ],
        out_specs=[
            pl.BlockSpec((blk, 128), lambda i: (i, 0)),
            pl.BlockSpec((blk, 128), lambda i: (i, 0)),
            pl.BlockSpec((1, 1), lambda i: (0, 0)),
            pl.BlockSpec((1, 1), lambda i: (0, 0)),
        ],
        out_shape=[
            jax.ShapeDtypeStruct((VP, 128), jnp.float32),
            jax.ShapeDtypeStruct((VP, 128), jnp.float32),
            jax.ShapeDtypeStruct((1, 1), jnp.float32),
            jax.ShapeDtypeStruct((1, 1), jnp.float32),
        ],
    )(embed, Ar, wrd, Ac, wcd, c1col, c0col, n1col, n0col, d1col, d0col)


def _tc_big0(flat2, deg2, G1, G0, embed, Av, wvd, Am, wmd, b2, Sn, Sd):
    blk = 2000

    def body(f_ref, d_ref, g1_ref, g0_ref, e_ref, av_ref, wvd_ref,
             am_ref, wmd_ref, b_ref, sn_ref, sd_ref, o_ref):
        e0 = e_ref[0:1, :]
        de = e_ref[1:2, :] - e0
        q0 = jnp.dot(e0, av_ref[...].T, preferred_element_type=jnp.float32)
        qd = jnp.dot(de, av_ref[...].T, preferred_element_type=jnp.float32)
        p0m = jnp.dot(e0, am_ref[...].T, preferred_element_type=jnp.float32)
        pdm = jnp.dot(de, am_ref[...].T, preferred_element_type=jnp.float32)
        mvec = (p0m + (sn_ref[...] * (1.0 / E)) * pdm
                + (sd_ref[...] * (1.0 / E)) * wmd_ref[...] + b_ref[...])
        pre = (q0 + f_ref[...] * qd + d_ref[...] * wvd_ref[...]
               + g1_ref[...] + g0_ref[...] + mvec)
        o_ref[...] = jnp.where(pre >= 0.0, pre, 0.01 * pre)

    return pl.pallas_call(
        body,
        grid=(E // blk,),
        in_specs=[
            pl.BlockSpec((blk, 1), lambda i: (i, 0)),
            pl.BlockSpec((blk, 1), lambda i: (i, 0)),
            pl.BlockSpec((blk, 128), lambda i: (i, 0)),
            pl.BlockSpec((blk, 128), lambda i: (i, 0)),
            pl.BlockSpec((2, 128), lambda i: (0, 0)),
            pl.BlockSpec((128, 128), lambda i: (0, 0)),
            pl.BlockSpec((1, 128), lambda i: (0, 0)),
            pl.BlockSpec((128, 128), lambda i: (0, 0)),
            pl.BlockSpec((1, 128), lambda i: (0, 0)),
            pl.BlockSpec((1, 128), lambda i: (0, 0)),
            pl.BlockSpec((1, 1), lambda i: (0, 0)),
            pl.BlockSpec((1, 1), lambda i: (0, 0)),
        ],
        out_specs=pl.BlockSpec((blk, 128), lambda i: (i, 0)),
        out_shape=jax.ShapeDtypeStruct((E, 128), jnp.float32),
    )(flat2, deg2, G1, G0, embed, Av, wvd, Am, wmd, b2, Sn, Sd)


def _tc_small(Rp, Cp, c1col, c0col, Wr, Wc):
    D = Rp.shape[1]
    blk = 2048

    def body(r_ref, c_ref, c1_ref, c0_ref, wr_ref, wc_ref,
             rq_ref, cq_ref, cs_ref):
        i = pl.program_id(0)
        Rs = r_ref[...]
        Cs = c_ref[...]
        rq_ref[...] = jnp.dot(Rs, wr_ref[...].T, preferred_element_type=jnp.float32) * c1_ref[...]
        cq_ref[...] = jnp.dot(Cs, wc_ref[...].T, preferred_element_type=jnp.float32) * c0_ref[...]

        @pl.when(i == 0)
        def _():
            cs_ref[...] = jnp.zeros((1, D), jnp.float32)
        cs_ref[...] += jnp.sum(Rs, axis=0, keepdims=True)

    return pl.pallas_call(
        body,
        grid=(VP // blk,),
        in_specs=[
            pl.BlockSpec((blk, D), lambda i: (i, 0)),
            pl.BlockSpec((blk, D), lambda i: (i, 0)),
            pl.BlockSpec((blk, 1), lambda i: (i, 0)),
            pl.BlockSpec((blk, 1), lambda i: (i, 0)),
            pl.BlockSpec(Wr.shape, lambda i: (0, 0)),
            pl.BlockSpec(Wc.shape, lambda i: (0, 0)),
        ],
        out_specs=[
            pl.BlockSpec((blk, 128), lambda i: (i, 0)),
            pl.BlockSpec((blk, 128), lambda i: (i, 0)),
            pl.BlockSpec((1, D), lambda i: (0, 0)),
        ],
        out_shape=[
            jax.ShapeDtypeStruct((VP, 128), jnp.float32),
            jax.ShapeDtypeStruct((VP, 128), jnp.float32),
            jax.ShapeDtypeStruct((1, D), jnp.float32),
        ],
    )(Rp, Cp, c1col, c0col, Wr, Wc)


def _tc_big(v, G1, G0, Wv, colsum, Wm, b2):
    D = v.shape[1]
    blk = 2000

    def body(v_ref, g1_ref, g0_ref, wv_ref, cs_ref, wm_ref, b_ref, o_ref):
        mvec = jnp.dot(cs_ref[...] * (1.0 / E), wm_ref[...].T,
                       preferred_element_type=jnp.float32) + b_ref[...]
        pre = (jnp.dot(v_ref[...], wv_ref[...].T, preferred_element_type=jnp.float32)
               + g1_ref[...] + g0_ref[...] + mvec)
        o_ref[...] = jnp.where(pre >= 0.0, pre, 0.01 * pre)

    return pl.pallas_call(
        body,
        grid=(E // blk,),
        in_specs=[
            pl.BlockSpec((blk, D), lambda i: (i, 0)),
            pl.BlockSpec((blk, 128), lambda i: (i, 0)),
            pl.BlockSpec((blk, 128), lambda i: (i, 0)),
            pl.BlockSpec(Wv.shape, lambda i: (0, 0)),
            pl.BlockSpec((1, D), lambda i: (0, 0)),
            pl.BlockSpec(Wm.shape, lambda i: (0, 0)),
            pl.BlockSpec((1, 128), lambda i: (0, 0)),
        ],
        out_specs=pl.BlockSpec((blk, 128), lambda i: (i, 0)),
        out_shape=jax.ShapeDtypeStruct((E, 128), jnp.float32),
    )(v, G1, G0, Wv, colsum, Wm, b2)


def _tc_final(emp, W1, b1, W2, b2, W3p, b3p):
    blk = 2048

    def body(e_ref, w1_ref, b1_ref, w2_ref, b2_ref, w3_ref, b3_ref,
             lg_ref, cs_ref):
        i = pl.program_id(0)
        em = e_ref[...]
        h = jnp.maximum(jnp.dot(em, w1_ref[...].T, preferred_element_type=jnp.float32) + b1_ref[...], 0.0)
        h = jnp.maximum(jnp.dot(h, w2_ref[...].T, preferred_element_type=jnp.float32) + b2_ref[...], 0.0)
        lg_ref[...] = jnp.dot(h, w3_ref[...].T, preferred_element_type=jnp.float32) + b3_ref[...]

        @pl.when(i == 0)
        def _():
            cs_ref[...] = jnp.zeros((1, 128), jnp.float32)
        cs_ref[...] += jnp.sum(em, axis=0, keepdims=True)

    return pl.pallas_call(
        body,
        grid=(VP // blk,),
        in_specs=[
            pl.BlockSpec((blk, 128), lambda i: (i, 0)),
            pl.BlockSpec(W1.shape, lambda i: (0, 0)),
            pl.BlockSpec((1, 128), lambda i: (0, 0)),
            pl.BlockSpec(W2.shape, lambda i: (0, 0)),
            pl.BlockSpec((1, 128), lambda i: (0, 0)),
            pl.BlockSpec(W3p.shape, lambda i: (0, 0)),
            pl.BlockSpec((1, 128), lambda i: (0, 0)),
        ],
        out_specs=[
            pl.BlockSpec((blk, 128), lambda i: (i, 0)),
            pl.BlockSpec((1, 128), lambda i: (0, 0)),
        ],
        out_shape=[
            jax.ShapeDtypeStruct((VP, 128), jnp.float32),
            jax.ShapeDtypeStruct((1, 128), jnp.float32),
        ],
    )(emp, W1, b1, W2, b2, W3p, b3p)


def _tc_value(emcol, sfp, W1p, b1, W2, b2, W3p, b3p):
    def body(ec_ref, sf_ref, w1_ref, b1_ref, w2_ref, b2_ref, w3_ref, b3_ref, o_ref):
        x = jnp.concatenate([ec_ref[...] * (1.0 / V), sf_ref[...]], axis=1)
        h = jnp.maximum(jnp.dot(x, w1_ref[...].T, preferred_element_type=jnp.float32) + b1_ref[...], 0.0)
        h = jnp.maximum(jnp.dot(h, w2_ref[...].T, preferred_element_type=jnp.float32) + b2_ref[...], 0.0)
        o_ref[...] = jnp.dot(h, w3_ref[...].T, preferred_element_type=jnp.float32) + b3_ref[...]

    return pl.pallas_call(
        body,
        out_shape=jax.ShapeDtypeStruct((1, 128), jnp.float32),
    )(emcol, sfp, W1p, b1, W2, b2, W3p, b3p)


def kernel(indices, values, embed,
           ex0_W, ex0_b, ex1_W, ex1_b, ex2_W, ex2_b, ex3_W, ex3_b, ex4_W, ex4_b,
           ex5_W, ex5_b, ex6_W, ex6_b, ex7_W, ex7_b, ex8_W, ex8_b,
           cl_W1, cl_b1, cl_W2, cl_b2, cl_W3, cl_b3,
           vl_W1, vl_b1, vl_W2, vl_b2, vl_W3, vl_b3):
    exW = [ex0_W, ex1_W, ex2_W, ex3_W, ex4_W, ex5_W, ex6_W, ex7_W, ex8_W]
    exb = [ex0_b, ex1_b, ex2_b, ex3_b, ex4_b, ex5_b, ex6_b, ex7_b, ex8_b]

    ind0 = indices[0].astype(jnp.int32)
    ind1 = indices[1].astype(jnp.int32)
    flat = values[:, 0]

    histp, c1p, c0p, n1p, n0p = _sc_stats(ind0, ind1, flat)
    hist2, urep2, c1i, c0i, c1r, c0r, n1r, n0r = _tc_prep(histp, c1p, c0p, n1p, n0p)
    compact = _sc_compact(hist2.reshape(NBINS))
    degree, dg1p, dg0p = _sc_degree(ind1, ind0, compact, urep2.reshape(L))
    dg1, dg0 = _tc_prep2(dg1p, dg0p)
    m0, m1 = _tc_max(ind0.reshape(E // 2000, 1, 2000), ind1.reshape(E // 2000, 1, 2000))

    c1col = c1i.reshape(VP, 1)
    c0col = c0i.reshape(VP, 1)

    W0 = exW[0]
    Wv0, Wr0, Wc0, Wm0 = W0[:, :129], W0[:, 129:258], W0[:, 258:387], W0[:, 387:516]
    Av, wvd = Wv0[:, :128], Wv0[:, 128].reshape(1, 128)
    Ar, wrd = Wr0[:, :128], Wr0[:, 128].reshape(1, 128)
    Ac, wcd = Wc0[:, :128], Wc0[:, 128].reshape(1, 128)
    Am, wmd = Wm0[:, :128], Wm0[:, 128].reshape(1, 128)
    Rq, Cq, Sn, Sd = _tc_small0(
        embed, Ar, wrd, Ac, wcd,
        c1r.reshape(VP, 1), c0r.reshape(VP, 1),
        n1r.reshape(VP, 1), n0r.reshape(VP, 1),
        dg1.reshape(VP, 1), dg0.reshape(VP, 1))
    G1 = _sc_gather(Rq, ind1)
    G0 = _sc_gather(Cq, ind0)
    v = _tc_big0(flat.reshape(E, 1), degree.reshape(E, 1), G1, G0, embed,
                 Av, wvd, Am, wmd, exb[0].reshape(1, 128), Sn, Sd)

    for i in range(1, 9):
        W = exW[i]
        Wv, Wr, Wc, Wm = W[:, :128], W[:, 128:256], W[:, 256:384], W[:, 384:512]
        seg = _sc_segsum(128)
        Rp = seg(v, ind1)
        Cp = seg(v, ind0)
        Rsum = jnp.concatenate([Rp[:VH], Rp[TAB:TAB + VH]], axis=0)
        Csum = jnp.concatenate([Cp[:VH], Cp[TAB:TAB + VH]], axis=0)
        Rq, Cq, colsum = _tc_small(Rsum, Csum, c1col, c0col, Wr, Wc)
        G1 = _sc_gather(Rq, ind1)
        G0 = _sc_gather(Cq, ind0)
        v = _tc_big(v, G1, G0, Wv, colsum, Wm, exb[i].reshape(1, 128))

    emp = _sc_segsum(128)(v, ind1)
    em = jnp.concatenate([emp[:VH], emp[TAB:TAB + VH]], axis=0)
    W3p = jnp.zeros((128, 128), jnp.float32).at[:2, :].set(cl_W3)
    b3p = jnp.zeros((1, 128), jnp.float32).at[0, :2].set(cl_b3)
    logits, emcol = _tc_final(em, cl_W1, cl_b1.reshape(1, 128),
                              cl_W2, cl_b2.reshape(1, 128), W3p, b3p)

    sfp = jnp.zeros((1, 16), jnp.float32)
    sfp = sfp.at[0, 0].set(float(E) / 100.0)
    sfp = sfp.at[0, 1].set(m0[0, 0] / 100.0)
    sfp = sfp.at[0, 2].set(m1[0, 0] / 100.0)
    vW1p = jnp.zeros((128, 144), jnp.float32)
    vW1p = vW1p.at[:, :128].set(vl_W1[:, :128])
    vW1p = vW1p.at[:, 128:131].set(vl_W1[:, 128:131])
    vW3p = jnp.zeros((128, 128), jnp.float32).at[:1, :].set(vl_W3)
    vb3p = jnp.zeros((1, 128), jnp.float32).at[0, :1].set(vl_b3)
    val = _tc_value(emcol, sfp, vW1p, vl_b1.reshape(1, 128),
                    vl_W2, vl_b2.reshape(1, 128), vW3p, vb3p)

    counts_out = logits[:V, :2].reshape(-1)
    return jnp.concatenate([counts_out, val[0, :1]])


# speedup vs baseline: 2.6735x; 1.0018x over previous
import functools

import jax
import jax.numpy as jnp
from jax import lax
from jax.experimental import pallas as pl
from jax.experimental.pallas import tpu as pltpu, tpu_sc as plsc

E = 320000
V = 10000
VP = 10240
NBINS = 20480
NC, NS, L = 2, 16, 16
NW = NC * NS
PERW = E // NW
CH = 200
NCH = PERW // CH

_mesh = plsc.VectorSubcoreMesh(core_axis_name="c", subcore_axis_name="s")
_scparams = pltpu.CompilerParams(needs_layout_passes=False)


def _zero_vmem(ref, n):
    def z(i, _):
        ref[pl.ds(i * L, L)] = jnp.zeros((L,), jnp.float32)
        return _
    lax.fori_loop(0, n // L, z, None)


@functools.partial(
    pl.kernel,
    out_type=[
        jax.ShapeDtypeStruct((NC, NBINS), jnp.float32),
        jax.ShapeDtypeStruct((NC, VP), jnp.float32),
        jax.ShapeDtypeStruct((NC, VP), jnp.float32),
        jax.ShapeDtypeStruct((NC, VP), jnp.float32),
        jax.ShapeDtypeStruct((NC, VP), jnp.float32),
    ],
    mesh=_mesh,
    compiler_params=_scparams,
    scratch_types=[
        pltpu.VMEM((PERW,), jnp.int32),
        pltpu.VMEM((PERW,), jnp.int32),
        pltpu.VMEM((PERW,), jnp.float32),
        pltpu.VMEM((NBINS,), jnp.float32),
        pltpu.VMEM((VP,), jnp.float32),
        pltpu.VMEM((VP,), jnp.float32),
        pltpu.VMEM((VP,), jnp.float32),
        pltpu.VMEM((VP,), jnp.float32),
        pltpu.VMEM((NBINS // NS,), jnp.float32),
        pltpu.VMEM((NBINS // NS,), jnp.float32),
        pltpu.VMEM_SHARED((NS, NBINS), jnp.float32),
    ],
)
def _sc_stats(i0_hbm, i1_hbm, f_hbm, hist_hbm, c1_hbm, c0_hbm, n1_hbm, n0_hbm,
              i0_v, i1_v, f_v, hist_v, c1_v, c0_v, n1_v, n0_v, acc_v, tmp_v,
              sh_h):
    cid = lax.axis_index("c")
    sid = lax.axis_index("s")
    wid = cid * NS + sid
    _zero_vmem(hist_v, NBINS)
    _zero_vmem(c1_v, VP)
    _zero_vmem(c0_v, VP)
    _zero_vmem(n1_v, VP)
    _zero_vmem(n0_v, VP)
    base = wid * PERW
    pltpu.sync_copy(i0_hbm.at[pl.ds(base, PERW)], i0_v)
    pltpu.sync_copy(i1_hbm.at[pl.ds(base, PERW)], i1_v)
    pltpu.sync_copy(f_hbm.at[pl.ds(base, PERW)], f_v)
    ones = jnp.ones((L,), jnp.float32)

    def body(j, _):
        i1 = i1_v[pl.ds(j * L, L)]
        i0 = i0_v[pl.ds(j * L, L)]
        f = f_v[pl.ds(j * L, L)]
        binv = jnp.where(f == 0.0, -i1, i1) + 9999
        plsc.addupdate_scatter(hist_v, [binv], ones)
        plsc.addupdate_scatter(c1_v, [i1], ones)
        plsc.addupdate_scatter(c0_v, [i0], ones)
        plsc.addupdate_scatter(n1_v, [i1], f)
        plsc.addupdate_scatter(n0_v, [i0], f)
        return _

    lax.fori_loop(0, PERW // L, body, None)

    def reduce_out(local_v, shared, out_ref, size):
        pltpu.sync_copy(local_v, shared.at[sid, pl.ds(0, size)])
        plsc.subcore_barrier()
        sl = size // NS
        rbase = sid * sl
        pltpu.sync_copy(shared.at[0, pl.ds(rbase, sl)], acc_v.at[pl.ds(0, sl)])

        def red(k, _):
            pltpu.sync_copy(shared.at[k, pl.ds(rbase, sl)], tmp_v.at[pl.ds(0, sl)])

            def addv(i, __):
                acc_v[pl.ds(i * L, L)] = acc_v[pl.ds(i * L, L)] + tmp_v[pl.ds(i * L, L)]
                return __
            lax.fori_loop(0, sl // L, addv, None)
            return _
        lax.fori_loop(1, NS, red, None)
        pltpu.sync_copy(acc_v.at[pl.ds(0, sl)], out_ref.at[cid, pl.ds(rbase, sl)])
        plsc.subcore_barrier()

    reduce_out(hist_v, sh_h, hist_hbm, NBINS)
    reduce_out(c1_v, sh_h, c1_hbm, VP)
    reduce_out(c0_v, sh_h, c0_hbm, VP)
    reduce_out(n1_v, sh_h, n1_hbm, VP)
    reduce_out(n0_v, sh_h, n0_hbm, VP)


@functools.partial(
    pl.kernel,
    out_type=jax.ShapeDtypeStruct((NBINS,), jnp.float32),
    mesh=_mesh,
    compiler_params=_scparams,
    scratch_types=[
        pltpu.VMEM((NBINS,), jnp.float32),
        pltpu.VMEM((NBINS,), jnp.float32),
    ],
)
def _sc_compact(hist_hbm, out_hbm, hist_v, comp_v):
    cid = lax.axis_index("c")
    sid = lax.axis_index("s")

    @pl.when(jnp.logical_and(cid == 0, sid == 0))
    def _():
        pltpu.sync_copy(hist_hbm, hist_v)
        _zero_vmem(comp_v, NBINS)

        def body(j, carry):
            v = hist_v[pl.ds(j * L, L)]
            mask = v > 0.0
            mi = mask.astype(jnp.int32)
            cs = plsc.cumsum(mi)
            pos = jnp.maximum(carry + cs - 1, 0)
            plsc.store_scatter(comp_v, [pos], v, mask=mask)
            return carry + jnp.sum(mi)

        lax.fori_loop(0, NBINS // L, body, jnp.int32(0))
        pltpu.sync_copy(comp_v, out_hbm)


@functools.partial(
    pl.kernel,
    out_type=[
        jax.ShapeDtypeStruct((E,), jnp.float32),
        jax.ShapeDtypeStruct((NC, VP), jnp.float32),
        jax.ShapeDtypeStruct((NC, VP), jnp.float32),
    ],
    mesh=_mesh,
    compiler_params=_scparams,
    scratch_types=[
        pltpu.VMEM((NBINS,), jnp.float32),
        pltpu.VMEM((L,), jnp.int32),
        pltpu.VMEM((PERW,), jnp.int32),
        pltpu.VMEM((PERW,), jnp.int32),
        pltpu.VMEM((PERW,), jnp.float32),
        pltpu.VMEM((VP,), jnp.float32),
        pltpu.VMEM((VP,), jnp.float32),
        pltpu.VMEM((VP // NS,), jnp.float32),
        pltpu.VMEM((VP // NS,), jnp.float32),
        pltpu.VMEM_SHARED((NS, VP), jnp.float32),
    ],
)
def _sc_degree(i1_hbm, i0_hbm, comp_hbm, urep_hbm,
               deg_hbm, dg1_hbm, dg0_hbm,
               comp_v, u_v, i1_v, i0_v, deg_v, dg1_v, dg0_v, acc_v, tmp_v, sh):
    cid = lax.axis_index("c")
    sid = lax.axis_index("s")
    wid = cid * NS + sid
    base = wid * PERW
    pltpu.sync_copy(comp_hbm, comp_v)
    pltpu.sync_copy(urep_hbm, u_v)
    pltpu.sync_copy(i1_hbm.at[pl.ds(base, PERW)], i1_v)
    pltpu.sync_copy(i0_hbm.at[pl.ds(base, PERW)], i0_v)
    _zero_vmem(dg1_v, VP)
    _zero_vmem(dg0_v, VP)

    def body(j, _):
        i1 = i1_v[pl.ds(j * L, L)]
        i0 = i0_v[pl.ds(j * L, L)]
        um = u_v[...]
        ic = jnp.maximum(jnp.minimum(i1, um - 1), 0)
        d = plsc.load_gather(comp_v, [ic])
        deg_v[pl.ds(j * L, L)] = d
        plsc.addupdate_scatter(dg1_v, [i1], d)
        plsc.addupdate_scatter(dg0_v, [i0], d)
        return _

    lax.fori_loop(0, PERW // L, body, None)
    pltpu.sync_copy(deg_v, deg_hbm.at[pl.ds(base, PERW)])

    def reduce_out(local_v, out_ref):
        pltpu.sync_copy(local_v, sh.at[sid])
        plsc.subcore_barrier()
        sl = VP // NS
        rbase = sid * sl
        pltpu.sync_copy(sh.at[0, pl.ds(rbase, sl)], acc_v)

        def red(k, _):
            pltpu.sync_copy(sh.at[k, pl.ds(rbase, sl)], tmp_v)

            def addv(i, __):
                acc_v[pl.ds(i * L, L)] = acc_v[pl.ds(i * L, L)] + tmp_v[pl.ds(i * L, L)]
                return __
            lax.fori_loop(0, sl // L, addv, None)
            return _
        lax.fori_loop(1, NS, red, None)
        pltpu.sync_copy(acc_v, out_ref.at[cid, pl.ds(rbase, sl)])
        plsc.subcore_barrier()

    reduce_out(dg1_v, dg1_hbm)
    reduce_out(dg0_v, dg0_hbm)


VH = VP // 2
TAB = VH + 128
RPT = TAB // NS
CHS = 160
PERC = E // NS


@functools.lru_cache(maxsize=None)
def _sc_segsum(D):
    @functools.partial(
        pl.kernel,
        out_type=[
            jax.ShapeDtypeStruct((NC * TAB, D), jnp.float32),
            jax.ShapeDtypeStruct((NC * TAB, D), jnp.float32),
        ],
        mesh=_mesh,
        compiler_params=_scparams,
        scratch_types=[
            pltpu.VMEM((CHS,), jnp.int32),
            pltpu.VMEM((CHS,), jnp.int32),
            pltpu.VMEM((CHS,), jnp.int32),
            pltpu.VMEM((CHS,), jnp.int32),
            pltpu.VMEM((CHS, D), jnp.float32),
            pltpu.VMEM((CHS, D), jnp.float32),
            pltpu.VMEM((RPT, D), jnp.float32),
            pltpu.VMEM_SHARED((TAB, D), jnp.float32),
            pltpu.SemaphoreType.DMA,
            pltpu.SemaphoreType.DMA,
        ],
    )
    def k(x_hbm, seg1_hbm, seg0_hbm, out1_hbm, out0_hbm,
          seg_a, seg_b, lidx_a, lidx_b,
          x_a, x_b, zbuf_v, table, sem_a, sem_b):
        bufs = (seg_a, seg_b, lidx_a, lidx_b, x_a, x_b, zbuf_v, table,
                sem_a, sem_b)
        _segsum_phase(D, x_hbm, seg1_hbm, out1_hbm, bufs)
        plsc.subcore_barrier()
        _segsum_phase(D, x_hbm, seg0_hbm, out0_hbm, bufs)

    return k


def _segsum_phase(D, x_hbm, seg_hbm, out_hbm, bufs):
    seg_a, seg_b, lidx_a, lidx_b, x_a, x_b, zbuf_v, table, sem_a, sem_b = bufs
    cid = lax.axis_index("c")
    sid = lax.axis_index("s")
    lo = cid * VH
    nchs = PERC // CHS

    def z2(i, _):
        def z3(j, __):
            zbuf_v[i, pl.ds(j * L, L)] = jnp.zeros((L,), jnp.float32)
            return __
        lax.fori_loop(0, D // L, z3, None)
        return _
    lax.fori_loop(0, RPT, z2, None)
    pltpu.sync_copy(zbuf_v, table.at[pl.ds(sid * RPT, RPT)])
    plsc.subcore_barrier()

    def issue(ch, segbuf, xbuf, sem):
        base = sid * PERC + ch * CHS
        pltpu.async_copy(seg_hbm.at[pl.ds(base, CHS)], segbuf, sem)
        pltpu.async_copy(x_hbm.at[pl.ds(base, CHS)], xbuf, sem)

    def wait(ch, segbuf, xbuf, sem):
        base = sid * PERC + ch * CHS
        pltpu.make_async_copy(seg_hbm.at[pl.ds(base, CHS)], segbuf, sem).wait()
        pltpu.make_async_copy(x_hbm.at[pl.ds(base, CHS)], xbuf, sem).wait()

    def scatter(segbuf, lidxbuf, xbuf):
        def tr(t, __):
            s = segbuf[pl.ds(t * L, L)] - lo
            oob = jnp.logical_or(s < 0, s >= VH)
            lidxbuf[pl.ds(t * L, L)] = jnp.where(oob, VH, s)
            return __
        lax.fori_loop(0, CHS // L, tr, None)
        pltpu.sync_copy(xbuf, table.at[lidxbuf], add=True)

    issue(0, seg_a, x_a, sem_a)

    def body(i, _):
        ch0 = 2 * i
        issue(ch0 + 1, seg_b, x_b, sem_b)
        wait(ch0, seg_a, x_a, sem_a)
        scatter(seg_a, lidx_a, x_a)
        issue(ch0 + 2, seg_a, x_a, sem_a)
        wait(ch0 + 1, seg_b, x_b, sem_b)
        scatter(seg_b, lidx_b, x_b)
        return _
    lax.fori_loop(0, (nchs - 1) // 2, body, None)
    wait(nchs - 1, seg_a, x_a, sem_a)
    scatter(seg_a, lidx_a, x_a)
    plsc.subcore_barrier()
    pltpu.sync_copy(table.at[pl.ds(sid * RPT, RPT)], zbuf_v)
    pltpu.sync_copy(zbuf_v, out_hbm.at[pl.ds(cid * TAB + sid * RPT, RPT)])


@functools.lru_cache(maxsize=None)
def _sc_segsum1(D):
    @functools.partial(
        pl.kernel,
        out_type=jax.ShapeDtypeStruct((NC * TAB, D), jnp.float32),
        mesh=_mesh,
        compiler_params=_scparams,
        scratch_types=[
            pltpu.VMEM((CHS,), jnp.int32),
            pltpu.VMEM((CHS,), jnp.int32),
            pltpu.VMEM((CHS,), jnp.int32),
            pltpu.VMEM((CHS,), jnp.int32),
            pltpu.VMEM((CHS, D), jnp.float32),
            pltpu.VMEM((CHS, D), jnp.float32),
            pltpu.VMEM((RPT, D), jnp.float32),
            pltpu.VMEM_SHARED((TAB, D), jnp.float32),
            pltpu.SemaphoreType.DMA,
            pltpu.SemaphoreType.DMA,
        ],
    )
    def k(x_hbm, seg_hbm, out_hbm, seg_a, seg_b, lidx_a, lidx_b,
          x_a, x_b, zbuf_v, table, sem_a, sem_b):
        bufs = (seg_a, seg_b, lidx_a, lidx_b, x_a, x_b, zbuf_v, table,
                sem_a, sem_b)
        _segsum_phase(D, x_hbm, seg_hbm, out_hbm, bufs)

    return k


CHG = 400
NCHG = PERW // CHG


@functools.partial(
    pl.kernel,
    out_type=[
        jax.ShapeDtypeStruct((E, 128), jnp.float32),
        jax.ShapeDtypeStruct((E, 128), jnp.float32),
    ],
    mesh=_mesh,
    compiler_params=_scparams,
    scratch_types=[
        pltpu.VMEM((CHG,), jnp.int32),
        pltpu.VMEM((CHG,), jnp.int32),
        pltpu.VMEM((CHG, 128), jnp.float32),
        pltpu.VMEM((CHG, 128), jnp.float32),
        pltpu.SemaphoreType.DMA,
        pltpu.SemaphoreType.DMA,
    ],
)
def _sc_gather(t1_hbm, i1_hbm, t0_hbm, i0_hbm, o1_hbm, o0_hbm,
               idx_a, idx_b, rows_a, rows_b, sem0, sem1):
    cid = lax.axis_index("c")
    sid = lax.axis_index("s")
    wid = cid * NS + sid
    base0 = wid * PERW
    idx_v = (idx_a, idx_b)
    rows_v = (rows_a, rows_b)
    sems = (sem0, sem1)

    def phase(table_hbm, idx_hbm, out_hbm):
        pltpu.sync_copy(idx_hbm.at[pl.ds(base0, CHG)], idx_a)
        h = pltpu.async_copy(table_hbm.at[idx_a], rows_a, sem0)
        for j in range(NCHG):
            b = j & 1
            h_next = None
            if j + 1 < NCHG:
                nb = (j + 1) & 1
                pltpu.sync_copy(idx_hbm.at[pl.ds(base0 + (j + 1) * CHG, CHG)],
                                idx_v[nb])
                h_next = pltpu.async_copy(table_hbm.at[idx_v[nb]],
                                          rows_v[nb], sems[nb])
            h.wait()
            pltpu.sync_copy(rows_v[b], out_hbm.at[pl.ds(base0 + j * CHG, CHG)])
            h = h_next

    phase(t1_hbm, i1_hbm, o1_hbm)
    phase(t0_hbm, i0_hbm, o0_hbm)


def _tc_prep(histp, c1p, c0p, n1p, n0p):
    def body(h_ref, c1_ref, c0_ref, n1_ref, n0_ref,
             hist_ref, urep_ref, c1i_ref, c0i_ref,
             c1r_ref, c0r_ref, n1r_ref, n0r_ref):
        h = h_ref[0] + h_ref[1]
        hist_ref[0, :] = h
        u = jnp.sum((h > 0.0).astype(jnp.int32))
        urep_ref[...] = jnp.full((1, L), u, jnp.int32)
        c1r_ref[0, :] = c1_ref[0] + c1_ref[1]
        c0r_ref[0, :] = c0_ref[0] + c0_ref[1]
        n1r_ref[0, :] = n1_ref[0] + n1_ref[1]
        n0r_ref[0, :] = n0_ref[0] + n0_ref[1]
        c1i_ref[0, :] = 1.0 / jnp.maximum(c1_ref[0] + c1_ref[1], 1.0)
        c0i_ref[0, :] = 1.0 / jnp.maximum(c0_ref[0] + c0_ref[1], 1.0)

    return pl.pallas_call(
        body,
        out_shape=[
            jax.ShapeDtypeStruct((1, NBINS), jnp.float32),
            jax.ShapeDtypeStruct((1, L), jnp.int32),
            jax.ShapeDtypeStruct((1, VP), jnp.float32),
            jax.ShapeDtypeStruct((1, VP), jnp.float32),
            jax.ShapeDtypeStruct((1, VP), jnp.float32),
            jax.ShapeDtypeStruct((1, VP), jnp.float32),
            jax.ShapeDtypeStruct((1, VP), jnp.float32),
            jax.ShapeDtypeStruct((1, VP), jnp.float32),
        ],
    )(histp, c1p, c0p, n1p, n0p)


def _tc_prep2(d1p, d0p):
    def body(a_ref, b_ref, o1_ref, o0_ref):
        o1_ref[0, :] = a_ref[0] + a_ref[1]
        o0_ref[0, :] = b_ref[0] + b_ref[1]

    return pl.pallas_call(
        body,
        out_shape=[
            jax.ShapeDtypeStruct((1, VP), jnp.float32),
            jax.ShapeDtypeStruct((1, VP), jnp.float32),
        ],
    )(d1p, d0p)


def _tc_max(i0r, i1r):
    nb = i0r.shape[0]

    def body(a_ref, b_ref, m0_ref, m1_ref):
        i = pl.program_id(0)

        @pl.when(i == 0)
        def _():
            m0_ref[...] = jnp.zeros((1, 1), jnp.float32)
            m1_ref[...] = jnp.zeros((1, 1), jnp.float32)
        bm0 = jnp.max(a_ref[...]).astype(jnp.float32)
        bm1 = jnp.max(b_ref[...]).astype(jnp.float32)
        m0_ref[...] = jnp.maximum(m0_ref[...], jnp.full((1, 1), bm0, jnp.float32))
        m1_ref[...] = jnp.maximum(m1_ref[...], jnp.full((1, 1), bm1, jnp.float32))

    return pl.pallas_call(
        body,
        grid=(nb,),
        in_specs=[
            pl.BlockSpec((1, 1, i0r.shape[2]), lambda i: (i, 0, 0)),
            pl.BlockSpec((1, 1, i1r.shape[2]), lambda i: (i, 0, 0)),
        ],
        out_specs=[
            pl.BlockSpec((1, 1), lambda i: (0, 0)),
            pl.BlockSpec((1, 1), lambda i: (0, 0)),
        ],
        out_shape=[
            jax.ShapeDtypeStruct((1, 1), jnp.float32),
            jax.ShapeDtypeStruct((1, 1), jnp.float32),
        ],
    )(i0r, i1r)


def _tc_small0(embed, Ar, wrd, Ac, wcd, c1col, c0col, n1col, n0col, d1col, d0col):
    blk = 2048

    def body(e_ref, ar_ref, wrd_ref, ac_ref, wcd_ref,
             c1_ref, c0_ref, n1_ref, n0_ref, d1_ref, d0_ref,
             rq_ref, cq_ref, sn_ref, sd_ref):
        i = pl.program_id(0)
        e0 = e_ref[0:1, :]
        de = e_ref[1:2, :] - e0
        p0r = jnp.dot(e0, ar_ref[...].T, preferred_element_type=jnp.float32)
        pdr = jnp.dot(de, ar_ref[...].T, preferred_element_type=jnp.float32)
        p0c = jnp.dot(e0, ac_ref[...].T, preferred_element_type=jnp.float32)
        pdc = jnp.dot(de, ac_ref[...].T, preferred_element_type=jnp.float32)
        c1 = c1_ref[...]
        c0 = c0_ref[...]
        n1 = n1_ref[...]
        n0 = n0_ref[...]
        d1 = d1_ref[...]
        d0 = d0_ref[...]
        rq_ref[...] = (c1 * p0r + n1 * pdr + d1 * wrd_ref[...]) / jnp.maximum(c1, 1.0)
        cq_ref[...] = (c0 * p0c + n0 * pdc + d0 * wcd_ref[...]) / jnp.maximum(c0, 1.0)

        @pl.when(i == 0)
        def _():
            sn_ref[...] = jnp.zeros((1, 1), jnp.float32)
            sd_ref[...] = jnp.zeros((1, 1), jnp.float32)
        sn_ref[...] += jnp.sum(n1, keepdims=True).reshape(1, 1)
        sd_ref[...] += jnp.sum(d1, keepdims=True).reshape(1, 1)

    return pl.pallas_call(
        body,
        grid=(VP // blk,),
        in_specs=[
            pl.BlockSpec((2, 128), lambda i: (0, 0)),
            pl.BlockSpec((128, 128), lambda i: (0, 0)),
            pl.BlockSpec((1, 128), lambda i: (0, 0)),
            pl.BlockSpec((128, 128), lambda i: (0, 0)),
            pl.BlockSpec((1, 128), lambda i: (0, 0)),
            pl.BlockSpec((blk, 1), lambda i: (i, 0)),
            pl.BlockSpec((blk, 1), lambda i: (i, 0)),
            pl.BlockSpec((blk, 1), lambda i: (i, 0)),
            pl.BlockSpec((blk, 1), lambda i: (i, 0)),
            pl.BlockSpec((blk, 1), lambda i: (i, 0)),
            pl.BlockSpec((blk, 1), lambda i: (i, 0)),
        ],
        out_specs=[
            pl.BlockSpec((blk, 128), lambda i: (i, 0)),
            pl.BlockSpec((blk, 128), lambda i: (i, 0)),
            pl.BlockSpec((1, 1), lambda i: (0, 0)),
            pl.BlockSpec((1, 1), lambda i: (0, 0)),
        ],
        out_shape=[
            jax.ShapeDtypeStruct((VP, 128), jnp.float32),
            jax.ShapeDtypeStruct((VP, 128), jnp.float32),
            jax.ShapeDtypeStruct((1, 1), jnp.float32),
            jax.ShapeDtypeStruct((1, 1), jnp.float32),
        ],
    )(embed, Ar, wrd, Ac, wcd, c1col, c0col, n1col, n0col, d1col, d0col)


def _tc_big0(flat2, deg2, G1, G0, embed, Av, wvd, Am, wmd, b2, Sn, Sd):
    blk = 2000

    def body(f_ref, d_ref, g1_ref, g0_ref, e_ref, av_ref, wvd_ref,
             am_ref, wmd_ref, b_ref, sn_ref, sd_ref, o_ref):
        e0 = e_ref[0:1, :]
        de = e_ref[1:2, :] - e0
        q0 = jnp.dot(e0, av_ref[...].T, preferred_element_type=jnp.float32)
        qd = jnp.dot(de, av_ref[...].T, preferred_element_type=jnp.float32)
        p0m = jnp.dot(e0, am_ref[...].T, preferred_element_type=jnp.float32)
        pdm = jnp.dot(de, am_ref[...].T, preferred_element_type=jnp.float32)
        mvec = (p0m + (sn_ref[...] * (1.0 / E)) * pdm
                + (sd_ref[...] * (1.0 / E)) * wmd_ref[...] + b_ref[...])
        pre = (q0 + f_ref[...] * qd + d_ref[...] * wvd_ref[...]
               + g1_ref[...] + g0_ref[...] + mvec)
        o_ref[...] = jnp.where(pre >= 0.0, pre, 0.01 * pre)

    return pl.pallas_call(
        body,
        grid=(E // blk,),
        in_specs=[
            pl.BlockSpec((blk, 1), lambda i: (i, 0)),
            pl.BlockSpec((blk, 1), lambda i: (i, 0)),
            pl.BlockSpec((blk, 128), lambda i: (i, 0)),
            pl.BlockSpec((blk, 128), lambda i: (i, 0)),
            pl.BlockSpec((2, 128), lambda i: (0, 0)),
            pl.BlockSpec((128, 128), lambda i: (0, 0)),
            pl.BlockSpec((1, 128), lambda i: (0, 0)),
            pl.BlockSpec((128, 128), lambda i: (0, 0)),
            pl.BlockSpec((1, 128), lambda i: (0, 0)),
            pl.BlockSpec((1, 128), lambda i: (0, 0)),
            pl.BlockSpec((1, 1), lambda i: (0, 0)),
            pl.BlockSpec((1, 1), lambda i: (0, 0)),
        ],
        out_specs=pl.BlockSpec((blk, 128), lambda i: (i, 0)),
        out_shape=jax.ShapeDtypeStruct((E, 128), jnp.float32),
    )(flat2, deg2, G1, G0, embed, Av, wvd, Am, wmd, b2, Sn, Sd)


def _tc_small(Rp, Cp, c1col, c0col, Wr, Wc):
    D = Rp.shape[1]
    blk = 2048

    def body(r_ref, c_ref, c1_ref, c0_ref, wr_ref, wc_ref,
             rq_ref, cq_ref, cs_ref):
        i = pl.program_id(0)
        Rs = r_ref[...]
        Cs = c_ref[...]
        rq_ref[...] = jnp.dot(Rs, wr_ref[...].T, preferred_element_type=jnp.float32) * c1_ref[...]
        cq_ref[...] = jnp.dot(Cs, wc_ref[...].T, preferred_element_type=jnp.float32) * c0_ref[...]

        @pl.when(i == 0)
        def _():
            cs_ref[...] = jnp.zeros((1, D), jnp.float32)
        cs_ref[...] += jnp.sum(Rs, axis=0, keepdims=True)

    return pl.pallas_call(
        body,
        grid=(VP // blk,),
        in_specs=[
            pl.BlockSpec((blk, D), lambda i: (i, 0)),
            pl.BlockSpec((blk, D), lambda i: (i, 0)),
            pl.BlockSpec((blk, 1), lambda i: (i, 0)),
            pl.BlockSpec((blk, 1), lambda i: (i, 0)),
            pl.BlockSpec(Wr.shape, lambda i: (0, 0)),
            pl.BlockSpec(Wc.shape, lambda i: (0, 0)),
        ],
        out_specs=[
            pl.BlockSpec((blk, 128), lambda i: (i, 0)),
            pl.BlockSpec((blk, 128), lambda i: (i, 0)),
            pl.BlockSpec((1, D), lambda i: (0, 0)),
        ],
        out_shape=[
            jax.ShapeDtypeStruct((VP, 128), jnp.float32),
            jax.ShapeDtypeStruct((VP, 128), jnp.float32),
            jax.ShapeDtypeStruct((1, D), jnp.float32),
        ],
    )(Rp, Cp, c1col, c0col, Wr, Wc)


def _tc_big(v, G1, G0, Wv, colsum, Wm, b2):
    D = v.shape[1]
    blk = 2000

    def body(v_ref, g1_ref, g0_ref, wv_ref, cs_ref, wm_ref, b_ref, o_ref):
        mvec = jnp.dot(cs_ref[...] * (1.0 / E), wm_ref[...].T,
                       preferred_element_type=jnp.float32) + b_ref[...]
        pre = (jnp.dot(v_ref[...], wv_ref[...].T, preferred_element_type=jnp.float32)
               + g1_ref[...] + g0_ref[...] + mvec)
        o_ref[...] = jnp.where(pre >= 0.0, pre, 0.01 * pre)

    return pl.pallas_call(
        body,
        grid=(E // blk,),
        in_specs=[
            pl.BlockSpec((blk, D), lambda i: (i, 0)),
            pl.BlockSpec((blk, 128), lambda i: (i, 0)),
            pl.BlockSpec((blk, 128), lambda i: (i, 0)),
            pl.BlockSpec(Wv.shape, lambda i: (0, 0)),
            pl.BlockSpec((1, D), lambda i: (0, 0)),
            pl.BlockSpec(Wm.shape, lambda i: (0, 0)),
            pl.BlockSpec((1, 128), lambda i: (0, 0)),
        ],
        out_specs=pl.BlockSpec((blk, 128), lambda i: (i, 0)),
        out_shape=jax.ShapeDtypeStruct((E, 128), jnp.float32),
    )(v, G1, G0, Wv, colsum, Wm, b2)


def _tc_final(emp, W1, b1, W2, b2, W3p, b3p):
    blk = 2048

    def body(e_ref, w1_ref, b1_ref, w2_ref, b2_ref, w3_ref, b3_ref,
             lg_ref, cs_ref):
        i = pl.program_id(0)
        em = e_ref[...]
        h = jnp.maximum(jnp.dot(em, w1_ref[...].T, preferred_element_type=jnp.float32) + b1_ref[...], 0.0)
        h = jnp.maximum(jnp.dot(h, w2_ref[...].T, preferred_element_type=jnp.float32) + b2_ref[...], 0.0)
        lg_ref[...] = jnp.dot(h, w3_ref[...].T, preferred_element_type=jnp.float32) + b3_ref[...]

        @pl.when(i == 0)
        def _():
            cs_ref[...] = jnp.zeros((1, 128), jnp.float32)
        cs_ref[...] += jnp.sum(em, axis=0, keepdims=True)

    return pl.pallas_call(
        body,
        grid=(VP // blk,),
        in_specs=[
            pl.BlockSpec((blk, 128), lambda i: (i, 0)),
            pl.BlockSpec(W1.shape, lambda i: (0, 0)),
            pl.BlockSpec((1, 128), lambda i: (0, 0)),
            pl.BlockSpec(W2.shape, lambda i: (0, 0)),
            pl.BlockSpec((1, 128), lambda i: (0, 0)),
            pl.BlockSpec(W3p.shape, lambda i: (0, 0)),
            pl.BlockSpec((1, 128), lambda i: (0, 0)),
        ],
        out_specs=[
            pl.BlockSpec((blk, 128), lambda i: (i, 0)),
            pl.BlockSpec((1, 128), lambda i: (0, 0)),
        ],
        out_shape=[
            jax.ShapeDtypeStruct((VP, 128), jnp.float32),
            jax.ShapeDtypeStruct((1, 128), jnp.float32),
        ],
    )(emp, W1, b1, W2, b2, W3p, b3p)


def _tc_value(emcol, sfp, W1p, b1, W2, b2, W3p, b3p):
    def body(ec_ref, sf_ref, w1_ref, b1_ref, w2_ref, b2_ref, w3_ref, b3_ref, o_ref):
        x = jnp.concatenate([ec_ref[...] * (1.0 / V), sf_ref[...]], axis=1)
        h = jnp.maximum(jnp.dot(x, w1_ref[...].T, preferred_element_type=jnp.float32) + b1_ref[...], 0.0)
        h = jnp.maximum(jnp.dot(h, w2_ref[...].T, preferred_element_type=jnp.float32) + b2_ref[...], 0.0)
        o_ref[...] = jnp.dot(h, w3_ref[...].T, preferred_element_type=jnp.float32) + b3_ref[...]

    return pl.pallas_call(
        body,
        out_shape=jax.ShapeDtypeStruct((1, 128), jnp.float32),
    )(emcol, sfp, W1p, b1, W2, b2, W3p, b3p)


def kernel(indices, values, embed,
           ex0_W, ex0_b, ex1_W, ex1_b, ex2_W, ex2_b, ex3_W, ex3_b, ex4_W, ex4_b,
           ex5_W, ex5_b, ex6_W, ex6_b, ex7_W, ex7_b, ex8_W, ex8_b,
           cl_W1, cl_b1, cl_W2, cl_b2, cl_W3, cl_b3,
           vl_W1, vl_b1, vl_W2, vl_b2, vl_W3, vl_b3):
    exW = [ex0_W, ex1_W, ex2_W, ex3_W, ex4_W, ex5_W, ex6_W, ex7_W, ex8_W]
    exb = [ex0_b, ex1_b, ex2_b, ex3_b, ex4_b, ex5_b, ex6_b, ex7_b, ex8_b]

    ind0 = indices[0].astype(jnp.int32)
    ind1 = indices[1].astype(jnp.int32)
    flat = values[:, 0]

    histp, c1p, c0p, n1p, n0p = _sc_stats(ind0, ind1, flat)
    hist2, urep2, c1i, c0i, c1r, c0r, n1r, n0r = _tc_prep(histp, c1p, c0p, n1p, n0p)
    compact = _sc_compact(hist2.reshape(NBINS))
    degree, dg1p, dg0p = _sc_degree(ind1, ind0, compact, urep2.reshape(L))
    dg1, dg0 = _tc_prep2(dg1p, dg0p)
    m0, m1 = _tc_max(ind0.reshape(E // 2000, 1, 2000), ind1.reshape(E // 2000, 1, 2000))

    c1col = c1i.reshape(VP, 1)
    c0col = c0i.reshape(VP, 1)

    W0 = exW[0]
    Wv0, Wr0, Wc0, Wm0 = W0[:, :129], W0[:, 129:258], W0[:, 258:387], W0[:, 387:516]
    Av, wvd = Wv0[:, :128], Wv0[:, 128].reshape(1, 128)
    Ar, wrd = Wr0[:, :128], Wr0[:, 128].reshape(1, 128)
    Ac, wcd = Wc0[:, :128], Wc0[:, 128].reshape(1, 128)
    Am, wmd = Wm0[:, :128], Wm0[:, 128].reshape(1, 128)
    Rq, Cq, Sn, Sd = _tc_small0(
        embed, Ar, wrd, Ac, wcd,
        c1r.reshape(VP, 1), c0r.reshape(VP, 1),
        n1r.reshape(VP, 1), n0r.reshape(VP, 1),
        dg1.reshape(VP, 1), dg0.reshape(VP, 1))
    G1, G0 = _sc_gather(Rq, ind1, Cq, ind0)
    v = _tc_big0(flat.reshape(E, 1), degree.reshape(E, 1), G1, G0, embed,
                 Av, wvd, Am, wmd, exb[0].reshape(1, 128), Sn, Sd)

    for i in range(1, 9):
        W = exW[i]
        Wv, Wr, Wc, Wm = W[:, :128], W[:, 128:256], W[:, 256:384], W[:, 384:512]
        Rp, Cp = _sc_segsum(128)(v, ind1, ind0)
        Rsum = jnp.concatenate([Rp[:VH], Rp[TAB:TAB + VH]], axis=0)
        Csum = jnp.concatenate([Cp[:VH], Cp[TAB:TAB + VH]], axis=0)
        Rq, Cq, colsum = _tc_small(Rsum, Csum, c1col, c0col, Wr, Wc)
        G1, G0 = _sc_gather(Rq, ind1, Cq, ind0)
        v = _tc_big(v, G1, G0, Wv, colsum, Wm, exb[i].reshape(1, 128))

    emp = _sc_segsum1(128)(v, ind1)
    em = jnp.concatenate([emp[:VH], emp[TAB:TAB + VH]], axis=0)
    W3p = jnp.zeros((128, 128), jnp.float32).at[:2, :].set(cl_W3)
    b3p = jnp.zeros((1, 128), jnp.float32).at[0, :2].set(cl_b3)
    logits, emcol = _tc_final(em, cl_W1, cl_b1.reshape(1, 128),
                              cl_W2, cl_b2.reshape(1, 128), W3p, b3p)

    sfp = jnp.zeros((1, 16), jnp.float32)
    sfp = sfp.at[0, 0].set(float(E) / 100.0)
    sfp = sfp.at[0, 1].set(m0[0, 0] / 100.0)
    sfp = sfp.at[0, 2].set(m1[0, 0] / 100.0)
    vW1p = jnp.zeros((128, 144), jnp.float32)
    vW1p = vW1p.at[:, :128].set(vl_W1[:, :128])
    vW1p = vW1p.at[:, 128:131].set(vl_W1[:, 128:131])
    vW3p = jnp.zeros((128, 128), jnp.float32).at[:1, :].set(vl_W3)
    vb3p = jnp.zeros((1, 128), jnp.float32).at[0, :1].set(vl_b3)
    val = _tc_value(emcol, sfp, vW1p, vl_b1.reshape(1, 128),
                    vl_W2, vl_b2.reshape(1, 128), vW3p, vb3p)

    counts_out = logits[:V, :2].reshape(-1)
    return jnp.concatenate([counts_out, val[0, :1]])
